# Initial kernel scaffold; baseline (speedup 1.0000x reference)
import jax
import jax.numpy as jnp
from jax.experimental import pallas as pl


def kernel(history_data, W_temp, b_temp, spatial_emb, enc_W1, enc_b1, enc_g1, enc_be1, enc_W2, enc_b2, enc_g2, enc_be2, W_h2e, b_h2e, Wq, bq, Wk, bk, Wv, bv, mWq, mbq, mWk, mbk, mWv, mbv, mWo, mbo, out_W1, out_b1, out_g, out_be, out_W2, out_b2, store):
    raise NotImplementedError("write your pallas kernel here")


# speedup vs baseline: 3.4107x; 3.4107x over previous
import functools
import math

import jax
import jax.numpy as jnp
from jax import lax
from jax.experimental import pallas as pl
from jax.experimental.pallas import tpu as pltpu
from jax.experimental.pallas import tpu_sc as plsc

_B, _L, _N, _C = 32, 12, 325, 3
_HOR, _OD = 12, 1
_TD, _SD = 64, 32
_F = _TD + _SD
_ED, _RD = 512, 512
_NH = 8
_DH = _RD // _NH
_KDOCS = 4096
_TOPK = 8
_EL = 3
_EPS = 1e-5
_R = _B * _N

_HI = lax.Precision.HIGHEST


def _ln(x, g, b):
    m = x.mean(-1, keepdims=True)
    v = ((x - m) ** 2).mean(-1, keepdims=True)
    return (x - m) * lax.rsqrt(v + _EPS) * g + b


def _dot(a, b):
    return jnp.dot(a, b, preferred_element_type=jnp.float32)


_QT1 = 400


def _enc_body(x_ref, sp_ref, Wt_ref, bt_ref, W1_ref, b1_ref, g1_ref, be1_ref,
              W2_ref, b2_ref, g2_ref, be2_ref, Wh_ref, bh_ref, Wq_ref, bq_ref,
              mWq_ref, mbq_ref, de_ref, q_ref, qm_ref):
    t = _dot(x_ref[...], Wt_ref[...]) + bt_ref[...]
    h = jnp.concatenate([t, sp_ref[...]], axis=-1)
    for i in range(_EL):
        h = jax.nn.relu(_ln(_dot(h, W1_ref[i]) + b1_ref[i], g1_ref[i], be1_ref[i]))
        h = jax.nn.relu(_ln(_dot(h, W2_ref[i]) + b2_ref[i], g2_ref[i], be2_ref[i]))
    de = _dot(h, Wh_ref[...]) + bh_ref[...]
    q = _dot(de, Wq_ref[...]) + bq_ref[...]
    de_ref[...] = de
    q_ref[...] = q
    qm_ref[...] = _dot(q, mWq_ref[...]) + mbq_ref[...]


def _encoder_call(x2, spf, W_temp, b_temp, enc_W1, enc_b1, enc_g1, enc_be1,
                  enc_W2, enc_b2, enc_g2, enc_be2, W_h2e, b_h2e, Wq, bq, mWq, mbq):
    grid = (_R // _QT1,)
    row = pl.BlockSpec((_QT1, None), lambda i: (i, 0))

    def full(a):
        return pl.BlockSpec(a.shape, lambda i: (0,) * a.ndim)

    row_specs = [pl.BlockSpec((_QT1, x2.shape[1]), lambda i: (i, 0)),
                 pl.BlockSpec((_QT1, spf.shape[1]), lambda i: (i, 0))]
    consts = [W_temp, b_temp, enc_W1, enc_b1, enc_g1, enc_be1,
              enc_W2, enc_b2, enc_g2, enc_be2, W_h2e, b_h2e, Wq, bq, mWq, mbq]
    out_spec = pl.BlockSpec((_QT1, _RD), lambda i: (i, 0))
    return pl.pallas_call(
        _enc_body,
        grid=grid,
        in_specs=row_specs + [full(c) for c in consts],
        out_specs=[out_spec, out_spec, out_spec],
        out_shape=[jax.ShapeDtypeStruct((_R, _RD), jnp.float32)] * 3,
    )(x2, spf, *consts)


def _kv_body(st_ref, Wk_ref, bk_ref, mWk_ref, mbk_ref,
             Wv_ref, bv_ref, mWv_ref, mbv_ref, kv_ref):
    s = st_ref[...]
    kk = _dot(_dot(s, Wk_ref[...]) + bk_ref[...], mWk_ref[...]) + mbk_ref[...]
    vv = _dot(_dot(s, Wv_ref[...]) + bv_ref[...], mWv_ref[...]) + mbv_ref[...]
    kv_ref[...] = jnp.concatenate([kk, vv], axis=-1)


def _kv_call(store, Wk, bk, mWk, mbk, Wv, bv, mWv, mbv):
    T = 512
    grid = (_KDOCS // T,)

    def full(a):
        return pl.BlockSpec(a.shape, lambda i: (0,) * a.ndim)

    consts = [Wk, bk, mWk, mbk, Wv, bv, mWv, mbv]
    return pl.pallas_call(
        _kv_body,
        grid=grid,
        in_specs=[pl.BlockSpec((T, _RD), lambda i: (i, 0))] + [full(c) for c in consts],
        out_specs=pl.BlockSpec((T, 2 * _RD), lambda i: (i, 0)),
        out_shape=jax.ShapeDtypeStruct((_KDOCS, 2 * _RD), jnp.float32),
    )(store, *consts)


_QT3 = 208


def _topk_body(q_ref, st_ref, idx_ref):
    s = lax.dot_general(q_ref[...], st_ref[...], (((1,), (1,)), ((), ())),
                        preferred_element_type=jnp.float32)
    cols = lax.broadcasted_iota(jnp.int32, (_QT3, _KDOCS), 1)
    outs = []
    for _ in range(_TOPK):
        m = jnp.max(s, axis=1, keepdims=True)
        cand = jnp.where(s >= m, cols, _KDOCS)
        cj = jnp.min(cand, axis=1, keepdims=True)
        outs.append(cj)
        s = jnp.where(cols == cj, -jnp.inf, s)
    idx_ref[...] = jnp.concatenate(outs, axis=1)


def _topk_call(q, store):
    grid = (_R // _QT3,)
    return pl.pallas_call(
        _topk_body,
        grid=grid,
        in_specs=[pl.BlockSpec((_QT3, _RD), lambda i: (i, 0)),
                  pl.BlockSpec((_KDOCS, _RD), lambda i: (0, 0))],
        out_specs=pl.BlockSpec((_QT3, _TOPK), lambda i: (i, 0)),
        out_shape=jax.ShapeDtypeStruct((_R, _TOPK), jnp.int32),
    )(q, store)


_NW = 32
_BT = _R * _TOPK
_BPW = _BT // _NW
_CH = 104


@functools.cache
def _build_sc_gather():
    @functools.partial(
        pl.kernel,
        mesh=plsc.VectorSubcoreMesh(core_axis_name="c", subcore_axis_name="s"),
        out_type=jax.ShapeDtypeStruct((_BT, 2 * _RD), jnp.float32),
        scratch_types=[
            pltpu.VMEM((_CH,), jnp.int32),
            pltpu.VMEM((_CH, 2 * _RD), jnp.float32),
            pltpu.SemaphoreType.DMA,
        ],
    )
    def _sc_gather(table_hbm, idx_hbm, out_hbm, idx_v, rows_v, sem):
        wid = lax.axis_index("s") * 2 + lax.axis_index("c")

        def body(i, carry):
            base = wid * _BPW + i * _CH
            pltpu.sync_copy(idx_hbm.at[pl.ds(base, _CH)], idx_v)
            pltpu.async_copy(table_hbm.at[idx_v], rows_v, sem).wait()
            pltpu.sync_copy(rows_v, out_hbm.at[pl.ds(base, _CH)])
            return carry

        lax.fori_loop(0, _BPW // _CH, body, 0)

    return _sc_gather


def _gather_call(kv, idx_flat):
    return _build_sc_gather()(kv, idx_flat)


_QT5 = 200


def _att_body(de_ref, qm_ref, kv_ref, mWo_ref, mbo_ref, W1_ref, b1_ref,
              g_ref, be_ref, W2_ref, b2_ref, y_ref):
    qm = qm_ref[...]
    kv = kv_ref[...]
    kg = kv[:, :, :_RD]
    vg = kv[:, :, _RD:]
    p = kg * qm[:, None, :]
    scs = [jnp.sum(p[:, :, h * _DH:(h + 1) * _DH], axis=-1) for h in range(_NH)]
    s = jnp.stack(scs, axis=-1) * (1.0 / math.sqrt(_DH))
    m = jnp.max(s, axis=1, keepdims=True)
    e = jnp.exp(s - m)
    a = e / jnp.sum(e, axis=1, keepdims=True)
    ab = jnp.concatenate(
        [jnp.broadcast_to(a[:, :, h:h + 1], (_QT5, _TOPK, _DH)) for h in range(_NH)],
        axis=-1)
    o = jnp.sum(ab * vg, axis=1)
    retr = _dot(o, mWo_ref[...]) + mbo_ref[...]
    comb = jnp.concatenate([de_ref[...], retr], axis=-1)
    z = jax.nn.relu(_ln(_dot(comb, W1_ref[...]) + b1_ref[...], g_ref[...], be_ref[...]))
    y_ref[...] = _dot(z, W2_ref[...]) + b2_ref[...]


def _att_call(de, qm, kvg, mWo, mbo, out_W1, out_b1, out_g, out_be, out_W2, out_b2):
    grid = (_R // _QT5,)

    def full(a):
        return pl.BlockSpec(a.shape, lambda i: (0,) * a.ndim)

    consts = [mWo, mbo, out_W1, out_b1, out_g, out_be, out_W2, out_b2]
    return pl.pallas_call(
        _att_body,
        grid=grid,
        in_specs=[pl.BlockSpec((_QT5, _RD), lambda i: (i, 0)),
                  pl.BlockSpec((_QT5, _RD), lambda i: (i, 0)),
                  pl.BlockSpec((_QT5, _TOPK, 2 * _RD), lambda i: (i, 0, 0))]
                 + [full(c) for c in consts],
        out_specs=pl.BlockSpec((_QT5, _HOR * _OD), lambda i: (i, 0)),
        out_shape=jax.ShapeDtypeStruct((_R, _HOR * _OD), jnp.float32),
    )(de, qm, kvg, *consts)


def kernel(history_data, W_temp, b_temp, spatial_emb, enc_W1, enc_b1, enc_g1,
           enc_be1, enc_W2, enc_b2, enc_g2, enc_be2, W_h2e, b_h2e, Wq, bq,
           Wk, bk, Wv, bv, mWq, mbq, mWk, mbk, mWv, mbv, mWo, mbo,
           out_W1, out_b1, out_g, out_be, out_W2, out_b2, store):
    x2 = history_data.transpose(0, 2, 1, 3).reshape(_R, _L * _C)
    spf = jnp.broadcast_to(spatial_emb[None], (_B, _N, _SD)).reshape(_R, _SD)

    r1 = lambda v: v.reshape(1, -1)
    r3 = lambda v: v.reshape(_EL, 1, -1)

    de, q, qm = _encoder_call(
        x2, spf, W_temp, r1(b_temp), enc_W1, r3(enc_b1), r3(enc_g1), r3(enc_be1),
        enc_W2, r3(enc_b2), r3(enc_g2), r3(enc_be2), W_h2e, r1(b_h2e),
        Wq, r1(bq), mWq, r1(mbq))

    kv = _kv_call(store, Wk, r1(bk), mWk, r1(mbk), Wv, r1(bv), mWv, r1(mbv))

    idx = _topk_call(q, store)

    g = _gather_call(kv, idx.reshape(_BT))

    y = _att_call(de, qm, g.reshape(_R, _TOPK, 2 * _RD), mWo, r1(mbo),
                  out_W1, r1(out_b1), r1(out_g), r1(out_be), out_W2, r1(out_b2))

    return y.reshape(_B, _N, _HOR, _OD).transpose(0, 2, 1, 3)


# speedup vs baseline: 3.5959x; 1.0543x over previous
import functools
import math

import jax
import jax.numpy as jnp
from jax import lax
from jax.experimental import pallas as pl
from jax.experimental.pallas import tpu as pltpu
from jax.experimental.pallas import tpu_sc as plsc

_B, _L, _N, _C = 32, 12, 325, 3
_HOR, _OD = 12, 1
_TD, _SD = 64, 32
_F = _TD + _SD
_ED, _RD = 512, 512
_NH = 8
_DH = _RD // _NH
_KDOCS = 4096
_TOPK = 8
_EL = 3
_EPS = 1e-5
_R = _B * _N

_HI = lax.Precision.HIGHEST


def _ln(x, g, b):
    m = x.mean(-1, keepdims=True)
    v = ((x - m) ** 2).mean(-1, keepdims=True)
    return (x - m) * lax.rsqrt(v + _EPS) * g + b


def _dot(a, b):
    return jnp.dot(a, b, preferred_element_type=jnp.float32)


_QT1 = 400


def _enc_body(x_ref, sp_ref, Wt_ref, bt_ref, W1_ref, b1_ref, g1_ref, be1_ref,
              W2_ref, b2_ref, g2_ref, be2_ref, Wh_ref, bh_ref, Wq_ref, bq_ref,
              mWq_ref, mbq_ref, de_ref, q_ref, qm_ref):
    t = _dot(x_ref[...], Wt_ref[...]) + bt_ref[...]
    h = jnp.concatenate([t, sp_ref[...]], axis=-1)
    for i in range(_EL):
        h = jax.nn.relu(_ln(_dot(h, W1_ref[i]) + b1_ref[i], g1_ref[i], be1_ref[i]))
        h = jax.nn.relu(_ln(_dot(h, W2_ref[i]) + b2_ref[i], g2_ref[i], be2_ref[i]))
    de = _dot(h, Wh_ref[...]) + bh_ref[...]
    q = _dot(de, Wq_ref[...]) + bq_ref[...]
    de_ref[...] = de
    q_ref[...] = q
    qm_ref[...] = _dot(q, mWq_ref[...]) + mbq_ref[...]


def _encoder_call(x2, spf, W_temp, b_temp, enc_W1, enc_b1, enc_g1, enc_be1,
                  enc_W2, enc_b2, enc_g2, enc_be2, W_h2e, b_h2e, Wq, bq, mWq, mbq):
    grid = (_R // _QT1,)
    row = pl.BlockSpec((_QT1, None), lambda i: (i, 0))

    def full(a):
        return pl.BlockSpec(a.shape, lambda i: (0,) * a.ndim)

    row_specs = [pl.BlockSpec((_QT1, x2.shape[1]), lambda i: (i, 0)),
                 pl.BlockSpec((_QT1, spf.shape[1]), lambda i: (i, 0))]
    consts = [W_temp, b_temp, enc_W1, enc_b1, enc_g1, enc_be1,
              enc_W2, enc_b2, enc_g2, enc_be2, W_h2e, b_h2e, Wq, bq, mWq, mbq]
    out_spec = pl.BlockSpec((_QT1, _RD), lambda i: (i, 0))
    return pl.pallas_call(
        _enc_body,
        grid=grid,
        in_specs=row_specs + [full(c) for c in consts],
        out_specs=[out_spec, out_spec, out_spec],
        out_shape=[jax.ShapeDtypeStruct((_R, _RD), jnp.float32)] * 3,
    )(x2, spf, *consts)


def _kv_body(st_ref, Wk_ref, bk_ref, mWk_ref, mbk_ref,
             Wv_ref, bv_ref, mWv_ref, mbv_ref, kv_ref):
    s = st_ref[...]
    kk = _dot(_dot(s, Wk_ref[...]) + bk_ref[...], mWk_ref[...]) + mbk_ref[...]
    vv = _dot(_dot(s, Wv_ref[...]) + bv_ref[...], mWv_ref[...]) + mbv_ref[...]
    k16 = lax.bitcast_convert_type(kk.astype(jnp.bfloat16), jnp.uint16).astype(jnp.uint32)
    v16 = lax.bitcast_convert_type(vv.astype(jnp.bfloat16), jnp.uint16).astype(jnp.uint32)
    kv_ref[...] = lax.bitcast_convert_type(k16 | (v16 << 16), jnp.int32)


def _kv_call(store, Wk, bk, mWk, mbk, Wv, bv, mWv, mbv):
    T = 512
    grid = (_KDOCS // T,)

    def full(a):
        return pl.BlockSpec(a.shape, lambda i: (0,) * a.ndim)

    consts = [Wk, bk, mWk, mbk, Wv, bv, mWv, mbv]
    return pl.pallas_call(
        _kv_body,
        grid=grid,
        in_specs=[pl.BlockSpec((T, _RD), lambda i: (i, 0))] + [full(c) for c in consts],
        out_specs=pl.BlockSpec((T, _RD), lambda i: (i, 0)),
        out_shape=jax.ShapeDtypeStruct((_KDOCS, _RD), jnp.int32),
    )(store, *consts)


_QT3 = 208


def _topk_body(q_ref, st_ref, idx_ref):
    s = lax.dot_general(q_ref[...], st_ref[...], (((1,), (1,)), ((), ())),
                        preferred_element_type=jnp.float32)
    cols = lax.broadcasted_iota(jnp.int32, (_QT3, _KDOCS), 1)
    outs = []
    for _ in range(_TOPK):
        m = jnp.max(s, axis=1, keepdims=True)
        cand = jnp.where(s >= m, cols, _KDOCS)
        cj = jnp.min(cand, axis=1, keepdims=True)
        outs.append(cj)
        s = jnp.where(cols == cj, -jnp.inf, s)
    idx_ref[...] = jnp.concatenate(outs, axis=1)


def _topk_call(q, store):
    grid = (_R // _QT3,)
    return pl.pallas_call(
        _topk_body,
        grid=grid,
        in_specs=[pl.BlockSpec((_QT3, _RD), lambda i: (i, 0)),
                  pl.BlockSpec((_KDOCS, _RD), lambda i: (0, 0))],
        out_specs=pl.BlockSpec((_QT3, _TOPK), lambda i: (i, 0)),
        out_shape=jax.ShapeDtypeStruct((_R, _TOPK), jnp.int32),
    )(q, store)


_NW = 32
_BT = _R * _TOPK
_BPW = _BT // _NW
_CH = 104


_NCH = _BPW // _CH


@functools.cache
def _build_sc_gather():
    @functools.partial(
        pl.kernel,
        mesh=plsc.VectorSubcoreMesh(core_axis_name="c", subcore_axis_name="s"),
        out_type=jax.ShapeDtypeStruct((_BT, _RD), jnp.int32),
        scratch_types=[
            pltpu.VMEM((_BPW,), jnp.int32),
            pltpu.VMEM((2, _CH, _RD), jnp.int32),
            pltpu.SemaphoreType.DMA,
            pltpu.SemaphoreType.DMA,
            pltpu.SemaphoreType.DMA,
            pltpu.SemaphoreType.DMA,
        ],
    )
    def _sc_gather(table_hbm, idx_hbm, out_hbm, idx_v, rows_v, g0, g1, w0, w1):
        wid = lax.axis_index("s") * 2 + lax.axis_index("c")
        base = wid * _BPW
        pltpu.sync_copy(idx_hbm.at[pl.ds(base, _BPW)], idx_v)
        gsem = (g0, g1)
        wsem = (w0, w1)

        def g_desc(i, b):
            return pltpu.make_async_copy(
                table_hbm.at[idx_v.at[pl.ds(i * _CH, _CH)]], rows_v.at[b], gsem[b])

        def w_desc(i, b):
            return pltpu.make_async_copy(
                rows_v.at[b], out_hbm.at[pl.ds(base + i * _CH, _CH)], wsem[b])

        g_desc(0, 0).start()

        def body(j, carry):
            i0 = 2 * j
            i1 = 2 * j + 1
            i2 = 2 * j + 2
            g_desc(i1, 1).start()
            g_desc(i0, 0).wait()
            w_desc(i0, 0).start()
            w_desc(i0, 0).wait()
            g_desc(i2, 0).start()
            g_desc(i1, 1).wait()
            w_desc(i1, 1).start()
            w_desc(i1, 1).wait()
            return carry

        lax.fori_loop(0, (_NCH - 1) // 2, body, 0)
        last = _NCH - 1
        g_desc(last, 0).wait()
        w_desc(last, 0).start()
        w_desc(last, 0).wait()

    return _sc_gather


def _gather_call(kv, idx_flat):
    return _build_sc_gather()(kv, idx_flat)


_QT5 = 200


def _att_body(de_ref, qm_ref, kv_ref, mWo_ref, mbo_ref, W1_ref, b1_ref,
              g_ref, be_ref, W2_ref, b2_ref, y_ref):
    qm = qm_ref[...]
    u = lax.bitcast_convert_type(kv_ref[...], jnp.uint32)
    kg = lax.bitcast_convert_type(
        (u & 0xffff).astype(jnp.uint16), jnp.bfloat16).astype(jnp.float32)
    vg = lax.bitcast_convert_type(
        (u >> 16).astype(jnp.uint16), jnp.bfloat16).astype(jnp.float32)
    p = kg * qm[:, None, :]
    scs = [jnp.sum(p[:, :, h * _DH:(h + 1) * _DH], axis=-1) for h in range(_NH)]
    s = jnp.stack(scs, axis=-1) * (1.0 / math.sqrt(_DH))
    m = jnp.max(s, axis=1, keepdims=True)
    e = jnp.exp(s - m)
    a = e / jnp.sum(e, axis=1, keepdims=True)
    ab = jnp.concatenate(
        [jnp.broadcast_to(a[:, :, h:h + 1], (_QT5, _TOPK, _DH)) for h in range(_NH)],
        axis=-1)
    o = jnp.sum(ab * vg, axis=1)
    retr = _dot(o, mWo_ref[...]) + mbo_ref[...]
    comb = jnp.concatenate([de_ref[...], retr], axis=-1)
    z = jax.nn.relu(_ln(_dot(comb, W1_ref[...]) + b1_ref[...], g_ref[...], be_ref[...]))
    y_ref[...] = _dot(z, W2_ref[...]) + b2_ref[...]


def _att_call(de, qm, kvg, mWo, mbo, out_W1, out_b1, out_g, out_be, out_W2, out_b2):
    grid = (_R // _QT5,)

    def full(a):
        return pl.BlockSpec(a.shape, lambda i: (0,) * a.ndim)

    consts = [mWo, mbo, out_W1, out_b1, out_g, out_be, out_W2, out_b2]
    return pl.pallas_call(
        _att_body,
        grid=grid,
        in_specs=[pl.BlockSpec((_QT5, _RD), lambda i: (i, 0)),
                  pl.BlockSpec((_QT5, _RD), lambda i: (i, 0)),
                  pl.BlockSpec((_QT5, _TOPK, _RD), lambda i: (i, 0, 0))]
                 + [full(c) for c in consts],
        out_specs=pl.BlockSpec((_QT5, _HOR * _OD), lambda i: (i, 0)),
        out_shape=jax.ShapeDtypeStruct((_R, _HOR * _OD), jnp.float32),
    )(de, qm, kvg, *consts)


def kernel(history_data, W_temp, b_temp, spatial_emb, enc_W1, enc_b1, enc_g1,
           enc_be1, enc_W2, enc_b2, enc_g2, enc_be2, W_h2e, b_h2e, Wq, bq,
           Wk, bk, Wv, bv, mWq, mbq, mWk, mbk, mWv, mbv, mWo, mbo,
           out_W1, out_b1, out_g, out_be, out_W2, out_b2, store):
    x2 = history_data.transpose(0, 2, 1, 3).reshape(_R, _L * _C)
    spf = jnp.broadcast_to(spatial_emb[None], (_B, _N, _SD)).reshape(_R, _SD)

    r1 = lambda v: v.reshape(1, -1)
    r3 = lambda v: v.reshape(_EL, 1, -1)

    de, q, qm = _encoder_call(
        x2, spf, W_temp, r1(b_temp), enc_W1, r3(enc_b1), r3(enc_g1), r3(enc_be1),
        enc_W2, r3(enc_b2), r3(enc_g2), r3(enc_be2), W_h2e, r1(b_h2e),
        Wq, r1(bq), mWq, r1(mbq))

    kv = _kv_call(store, Wk, r1(bk), mWk, r1(mbk), Wv, r1(bv), mWv, r1(mbv))

    idx = _topk_call(q, store)

    g = _gather_call(kv, idx.reshape(_BT))

    y = _att_call(de, qm, g.reshape(_R, _TOPK, _RD), mWo, r1(mbo),
                  out_W1, r1(out_b1), r1(out_g), r1(out_be), out_W2, r1(out_b2))

    return y.reshape(_B, _N, _HOR, _OD).transpose(0, 2, 1, 3)


# speedup vs baseline: 3.6062x; 1.0029x over previous
import functools
import math

import jax
import jax.numpy as jnp
from jax import lax
from jax.experimental import pallas as pl
from jax.experimental.pallas import tpu as pltpu
from jax.experimental.pallas import tpu_sc as plsc

_B, _L, _N, _C = 32, 12, 325, 3
_HOR, _OD = 12, 1
_TD, _SD = 64, 32
_F = _TD + _SD
_ED, _RD = 512, 512
_NH = 8
_DH = _RD // _NH
_KDOCS = 4096
_TOPK = 8
_EL = 3
_EPS = 1e-5
_R = _B * _N

_HI = lax.Precision.HIGHEST


def _ln(x, g, b):
    m = x.mean(-1, keepdims=True)
    v = ((x - m) ** 2).mean(-1, keepdims=True)
    return (x - m) * lax.rsqrt(v + _EPS) * g + b


def _dot(a, b):
    return jnp.dot(a, b, preferred_element_type=jnp.float32)


_QT1 = 400


def _enc_body(x_ref, sp_ref, Wt_ref, bt_ref, W1_ref, b1_ref, g1_ref, be1_ref,
              W2_ref, b2_ref, g2_ref, be2_ref, Wh_ref, bh_ref, Wq_ref, bq_ref,
              mWq_ref, mbq_ref, de_ref, q_ref, qm_ref):
    t = _dot(x_ref[...], Wt_ref[...]) + bt_ref[...]
    h = jnp.concatenate([t, sp_ref[...]], axis=-1)
    for i in range(_EL):
        h = jax.nn.relu(_ln(_dot(h, W1_ref[i]) + b1_ref[i], g1_ref[i], be1_ref[i]))
        h = jax.nn.relu(_ln(_dot(h, W2_ref[i]) + b2_ref[i], g2_ref[i], be2_ref[i]))
    de = _dot(h, Wh_ref[...]) + bh_ref[...]
    q = _dot(de, Wq_ref[...]) + bq_ref[...]
    de_ref[...] = de
    q_ref[...] = q
    qm_ref[...] = _dot(q, mWq_ref[...]) + mbq_ref[...]


def _encoder_call(x2, spf, W_temp, b_temp, enc_W1, enc_b1, enc_g1, enc_be1,
                  enc_W2, enc_b2, enc_g2, enc_be2, W_h2e, b_h2e, Wq, bq, mWq, mbq):
    grid = (_R // _QT1,)
    row = pl.BlockSpec((_QT1, None), lambda i: (i, 0))

    def full(a):
        return pl.BlockSpec(a.shape, lambda i: (0,) * a.ndim)

    row_specs = [pl.BlockSpec((_QT1, x2.shape[1]), lambda i: (i, 0)),
                 pl.BlockSpec((_QT1, spf.shape[1]), lambda i: (i, 0))]
    consts = [W_temp, b_temp, enc_W1, enc_b1, enc_g1, enc_be1,
              enc_W2, enc_b2, enc_g2, enc_be2, W_h2e, b_h2e, Wq, bq, mWq, mbq]
    out_spec = pl.BlockSpec((_QT1, _RD), lambda i: (i, 0))
    return pl.pallas_call(
        _enc_body,
        grid=grid,
        in_specs=row_specs + [full(c) for c in consts],
        out_specs=[out_spec, out_spec, out_spec],
        out_shape=[jax.ShapeDtypeStruct((_R, _RD), jnp.float32)] * 3,
    )(x2, spf, *consts)


def _kv_body(st_ref, Wk_ref, bk_ref, mWk_ref, mbk_ref,
             Wv_ref, bv_ref, mWv_ref, mbv_ref, kv_ref):
    s = st_ref[...]
    kk = _dot(_dot(s, Wk_ref[...]) + bk_ref[...], mWk_ref[...]) + mbk_ref[...]
    vv = _dot(_dot(s, Wv_ref[...]) + bv_ref[...], mWv_ref[...]) + mbv_ref[...]
    k16 = lax.bitcast_convert_type(kk.astype(jnp.bfloat16), jnp.uint16).astype(jnp.uint32)
    v16 = lax.bitcast_convert_type(vv.astype(jnp.bfloat16), jnp.uint16).astype(jnp.uint32)
    kv_ref[...] = lax.bitcast_convert_type(k16 | (v16 << 16), jnp.int32)


def _kv_call(store, Wk, bk, mWk, mbk, Wv, bv, mWv, mbv):
    T = 512
    grid = (_KDOCS // T,)

    def full(a):
        return pl.BlockSpec(a.shape, lambda i: (0,) * a.ndim)

    consts = [Wk, bk, mWk, mbk, Wv, bv, mWv, mbv]
    return pl.pallas_call(
        _kv_body,
        grid=grid,
        in_specs=[pl.BlockSpec((T, _RD), lambda i: (i, 0))] + [full(c) for c in consts],
        out_specs=pl.BlockSpec((T, _RD), lambda i: (i, 0)),
        out_shape=jax.ShapeDtypeStruct((_KDOCS, _RD), jnp.int32),
    )(store, *consts)


_QT3 = 208


def _topk_body(q_ref, st_ref, idx_ref):
    s = lax.dot_general(q_ref[...], st_ref[...], (((1,), (1,)), ((), ())),
                        preferred_element_type=jnp.float32)
    cols = lax.broadcasted_iota(jnp.int32, (_QT3, _KDOCS), 1)
    outs = []
    for _ in range(_TOPK):
        m = jnp.max(s, axis=1, keepdims=True)
        cand = jnp.where(s >= m, cols, _KDOCS)
        cj = jnp.min(cand, axis=1, keepdims=True)
        outs.append(cj)
        s = jnp.where(cols == cj, -jnp.inf, s)
    idx_ref[...] = jnp.concatenate(outs, axis=1)


def _topk_call(q, store):
    grid = (_R // _QT3,)
    return pl.pallas_call(
        _topk_body,
        grid=grid,
        in_specs=[pl.BlockSpec((_QT3, _RD), lambda i: (i, 0)),
                  pl.BlockSpec((_KDOCS, _RD), lambda i: (0, 0))],
        out_specs=pl.BlockSpec((_QT3, _TOPK), lambda i: (i, 0)),
        out_shape=jax.ShapeDtypeStruct((_R, _TOPK), jnp.int32),
    )(q, store)


_NW = 32
_BT = _R * _TOPK
_BPW = _BT // _NW
_CH = 40
_NB = 5


_NCH = _BPW // _CH


@functools.cache
def _build_sc_gather():
    @functools.partial(
        pl.kernel,
        mesh=plsc.VectorSubcoreMesh(core_axis_name="c", subcore_axis_name="s"),
        out_type=jax.ShapeDtypeStruct((_BT, _RD), jnp.int32),
        scratch_types=[
            pltpu.VMEM((_BPW,), jnp.int32),
            pltpu.VMEM((_NB, _CH, _RD), jnp.int32),
        ] + [pltpu.SemaphoreType.DMA] * (2 * _NB),
    )
    def _sc_gather(table_hbm, idx_hbm, out_hbm, idx_v, rows_v, *sems):
        gsem = sems[:_NB]
        wsem = sems[_NB:]
        wid = lax.axis_index("s") * 2 + lax.axis_index("c")
        base = wid * _BPW
        pltpu.sync_copy(idx_hbm.at[pl.ds(base, _BPW)], idx_v)

        def g_desc(i, b):
            return pltpu.make_async_copy(
                table_hbm.at[idx_v.at[pl.ds(i * _CH, _CH)]], rows_v.at[b], gsem[b])

        def w_desc(i, b):
            return pltpu.make_async_copy(
                rows_v.at[b], out_hbm.at[pl.ds(base + i * _CH, _CH)], wsem[b])

        for b in range(_NB):
            g_desc(b, b).start()

        def body(j, carry):
            c0 = j * _NB
            for b in range(_NB):
                g_desc(c0 + b, b).wait()
                w_desc(c0 + b, b).start()
            for b in range(_NB):
                w_desc(c0 + b, b).wait()
                g_desc(c0 + _NB + b, b).start()
            return carry

        lax.fori_loop(0, _NCH // _NB - 1, body, 0)
        c0 = _NCH - _NB
        for b in range(_NB):
            g_desc(c0 + b, b).wait()
            w_desc(c0 + b, b).start()
        for b in range(_NB):
            w_desc(c0 + b, b).wait()

    return _sc_gather


def _gather_call(kv, idx_flat):
    return _build_sc_gather()(kv, idx_flat)


_QT5 = 200


def _att_body(de_ref, qm_ref, kv_ref, mWo_ref, mbo_ref, W1_ref, b1_ref,
              g_ref, be_ref, W2_ref, b2_ref, y_ref):
    qm = qm_ref[...]
    u = lax.bitcast_convert_type(kv_ref[...], jnp.uint32)
    kg = lax.bitcast_convert_type(
        (u & 0xffff).astype(jnp.uint16), jnp.bfloat16).astype(jnp.float32)
    vg = lax.bitcast_convert_type(
        (u >> 16).astype(jnp.uint16), jnp.bfloat16).astype(jnp.float32)
    p = kg * qm[:, None, :]
    scs = [jnp.sum(p[:, :, h * _DH:(h + 1) * _DH], axis=-1) for h in range(_NH)]
    s = jnp.stack(scs, axis=-1) * (1.0 / math.sqrt(_DH))
    m = jnp.max(s, axis=1, keepdims=True)
    e = jnp.exp(s - m)
    a = e / jnp.sum(e, axis=1, keepdims=True)
    ab = jnp.concatenate(
        [jnp.broadcast_to(a[:, :, h:h + 1], (_QT5, _TOPK, _DH)) for h in range(_NH)],
        axis=-1)
    o = jnp.sum(ab * vg, axis=1)
    retr = _dot(o, mWo_ref[...]) + mbo_ref[...]
    comb = jnp.concatenate([de_ref[...], retr], axis=-1)
    z = jax.nn.relu(_ln(_dot(comb, W1_ref[...]) + b1_ref[...], g_ref[...], be_ref[...]))
    y_ref[...] = _dot(z, W2_ref[...]) + b2_ref[...]


def _att_call(de, qm, kvg, mWo, mbo, out_W1, out_b1, out_g, out_be, out_W2, out_b2):
    grid = (_R // _QT5,)

    def full(a):
        return pl.BlockSpec(a.shape, lambda i: (0,) * a.ndim)

    consts = [mWo, mbo, out_W1, out_b1, out_g, out_be, out_W2, out_b2]
    return pl.pallas_call(
        _att_body,
        grid=grid,
        in_specs=[pl.BlockSpec((_QT5, _RD), lambda i: (i, 0)),
                  pl.BlockSpec((_QT5, _RD), lambda i: (i, 0)),
                  pl.BlockSpec((_QT5, _TOPK, _RD), lambda i: (i, 0, 0))]
                 + [full(c) for c in consts],
        out_specs=pl.BlockSpec((_QT5, _HOR * _OD), lambda i: (i, 0)),
        out_shape=jax.ShapeDtypeStruct((_R, _HOR * _OD), jnp.float32),
    )(de, qm, kvg, *consts)


def kernel(history_data, W_temp, b_temp, spatial_emb, enc_W1, enc_b1, enc_g1,
           enc_be1, enc_W2, enc_b2, enc_g2, enc_be2, W_h2e, b_h2e, Wq, bq,
           Wk, bk, Wv, bv, mWq, mbq, mWk, mbk, mWv, mbv, mWo, mbo,
           out_W1, out_b1, out_g, out_be, out_W2, out_b2, store):
    x2 = history_data.transpose(0, 2, 1, 3).reshape(_R, _L * _C)
    spf = jnp.broadcast_to(spatial_emb[None], (_B, _N, _SD)).reshape(_R, _SD)

    r1 = lambda v: v.reshape(1, -1)
    r3 = lambda v: v.reshape(_EL, 1, -1)

    de, q, qm = _encoder_call(
        x2, spf, W_temp, r1(b_temp), enc_W1, r3(enc_b1), r3(enc_g1), r3(enc_be1),
        enc_W2, r3(enc_b2), r3(enc_g2), r3(enc_be2), W_h2e, r1(b_h2e),
        Wq, r1(bq), mWq, r1(mbq))

    kv = _kv_call(store, Wk, r1(bk), mWk, r1(mbk), Wv, r1(bv), mWv, r1(mbv))

    idx = _topk_call(q, store)

    g = _gather_call(kv, idx.reshape(_BT))

    y = _att_call(de, qm, g.reshape(_R, _TOPK, _RD), mWo, r1(mbo),
                  out_W1, r1(out_b1), r1(out_g), r1(out_be), out_W2, r1(out_b2))

    return y.reshape(_B, _N, _HOR, _OD).transpose(0, 2, 1, 3)


# speedup vs baseline: 3.9374x; 1.0918x over previous
import functools
import math

import jax
import jax.numpy as jnp
from jax import lax
from jax.experimental import pallas as pl
from jax.experimental.pallas import tpu as pltpu
from jax.experimental.pallas import tpu_sc as plsc

_B, _L, _N, _C = 32, 12, 325, 3
_HOR, _OD = 12, 1
_TD, _SD = 64, 32
_F = _TD + _SD
_ED, _RD = 512, 512
_NH = 8
_DH = _RD // _NH
_KDOCS = 4096
_TOPK = 8
_EL = 3
_EPS = 1e-5
_R = _B * _N

_HI = lax.Precision.HIGHEST


def _ln(x, g, b):
    m = x.mean(-1, keepdims=True)
    v = ((x - m) ** 2).mean(-1, keepdims=True)
    return (x - m) * lax.rsqrt(v + _EPS) * g + b


def _dot(a, b):
    return jnp.dot(a, b, preferred_element_type=jnp.float32)


_QT1 = 400


def _enc_body(x_ref, sp_ref, Wt_ref, bt_ref, W1_ref, b1_ref, g1_ref, be1_ref,
              W2_ref, b2_ref, g2_ref, be2_ref, Wh_ref, bh_ref, Wq_ref, bq_ref,
              mWq_ref, mbq_ref, de_ref, q_ref, qm_ref):
    t = _dot(x_ref[...], Wt_ref[...]) + bt_ref[...]
    h = jnp.concatenate([t, sp_ref[...]], axis=-1)
    for i in range(_EL):
        h = jax.nn.relu(_ln(_dot(h, W1_ref[i]) + b1_ref[i], g1_ref[i], be1_ref[i]))
        h = jax.nn.relu(_ln(_dot(h, W2_ref[i]) + b2_ref[i], g2_ref[i], be2_ref[i]))
    de = _dot(h, Wh_ref[...]) + bh_ref[...]
    q = _dot(de, Wq_ref[...]) + bq_ref[...]
    de_ref[...] = de
    q_ref[...] = q
    qm_ref[...] = _dot(q, mWq_ref[...]) + mbq_ref[...]


def _encoder_call(x2, spf, W_temp, b_temp, enc_W1, enc_b1, enc_g1, enc_be1,
                  enc_W2, enc_b2, enc_g2, enc_be2, W_h2e, b_h2e, Wq, bq, mWq, mbq):
    grid = (_R // _QT1,)
    row = pl.BlockSpec((_QT1, None), lambda i: (i, 0))

    def full(a):
        return pl.BlockSpec(a.shape, lambda i: (0,) * a.ndim)

    row_specs = [pl.BlockSpec((_QT1, x2.shape[1]), lambda i: (i, 0)),
                 pl.BlockSpec((_QT1, spf.shape[1]), lambda i: (i, 0))]
    consts = [W_temp, b_temp, enc_W1, enc_b1, enc_g1, enc_be1,
              enc_W2, enc_b2, enc_g2, enc_be2, W_h2e, b_h2e, Wq, bq, mWq, mbq]
    out_spec = pl.BlockSpec((_QT1, _RD), lambda i: (i, 0))
    return pl.pallas_call(
        _enc_body,
        grid=grid,
        in_specs=row_specs + [full(c) for c in consts],
        out_specs=[out_spec, out_spec, out_spec],
        out_shape=[jax.ShapeDtypeStruct((_R, _RD), jnp.float32)] * 3,
    )(x2, spf, *consts)


def _kv_body(st_ref, Wk_ref, bk_ref, mWk_ref, mbk_ref,
             Wv_ref, bv_ref, mWv_ref, mbv_ref, kv_ref):
    s = st_ref[...]
    kk = _dot(_dot(s, Wk_ref[...]) + bk_ref[...], mWk_ref[...]) + mbk_ref[...]
    vv = _dot(_dot(s, Wv_ref[...]) + bv_ref[...], mWv_ref[...]) + mbv_ref[...]
    k16 = lax.bitcast_convert_type(kk.astype(jnp.bfloat16), jnp.uint16).astype(jnp.uint32)
    v16 = lax.bitcast_convert_type(vv.astype(jnp.bfloat16), jnp.uint16).astype(jnp.uint32)
    kv_ref[...] = lax.bitcast_convert_type(k16 | (v16 << 16), jnp.int32)


def _kv_call(store, Wk, bk, mWk, mbk, Wv, bv, mWv, mbv):
    T = 512
    grid = (_KDOCS // T,)

    def full(a):
        return pl.BlockSpec(a.shape, lambda i: (0,) * a.ndim)

    consts = [Wk, bk, mWk, mbk, Wv, bv, mWv, mbv]
    return pl.pallas_call(
        _kv_body,
        grid=grid,
        in_specs=[pl.BlockSpec((T, _RD), lambda i: (i, 0))] + [full(c) for c in consts],
        out_specs=pl.BlockSpec((T, _RD), lambda i: (i, 0)),
        out_shape=jax.ShapeDtypeStruct((_KDOCS, _RD), jnp.int32),
    )(store, *consts)


_QT3 = 208


def _topk_body(q_ref, st_ref, idx_ref):
    s = lax.dot_general(q_ref[...], st_ref[...], (((1,), (1,)), ((), ())),
                        preferred_element_type=jnp.float32)
    cols = lax.broadcasted_iota(jnp.int32, (_QT3, _KDOCS), 1)
    outs = []
    for _ in range(_TOPK):
        m = jnp.max(s, axis=1, keepdims=True)
        cand = jnp.where(s >= m, cols, _KDOCS)
        cj = jnp.min(cand, axis=1, keepdims=True)
        outs.append(cj)
        s = jnp.where(cols == cj, -jnp.inf, s)
    idx_ref[...] = jnp.concatenate(outs, axis=1)


def _topk_call(q, store):
    grid = (_R // _QT3,)
    return pl.pallas_call(
        _topk_body,
        grid=grid,
        in_specs=[pl.BlockSpec((_QT3, _RD), lambda i: (i, 0)),
                  pl.BlockSpec((_KDOCS, _RD), lambda i: (0, 0))],
        out_specs=pl.BlockSpec((_QT3, _TOPK), lambda i: (i, 0)),
        out_shape=jax.ShapeDtypeStruct((_R, _TOPK), jnp.int32),
    )(q, store)


_NW = 32
_BT = _R * _TOPK
_BTP = 83968
_BPW = _BTP // _NW
_CH = 32


_NCH = _BPW // _CH


@functools.cache
def _build_sc_gather():
    @functools.partial(
        pl.kernel,
        mesh=plsc.VectorSubcoreMesh(core_axis_name="c", subcore_axis_name="s"),
        out_type=jax.ShapeDtypeStruct((_BTP, _RD), jnp.int32),
        scratch_types=[
            pltpu.VMEM((_BPW,), jnp.int32),
            pltpu.VMEM((2, _CH, _RD), jnp.int32),
        ] + [pltpu.SemaphoreType.DMA] * 4,
    )
    def _sc_gather(table_hbm, idx_hbm, out_hbm, idx_v, rows_v, *sems):
        gsem = sems[:2]
        wsem = sems[2:]
        wid = lax.axis_index("s") * 2 + lax.axis_index("c")
        base = wid * _BPW
        pltpu.sync_copy(idx_hbm.at[pl.ds(base, _BPW)], idx_v)

        def g_start(i, b):
            for sub in range(_CH // 16):
                idx16 = idx_v[pl.ds(i * _CH + sub * 16, 16)]
                pltpu.make_async_copy(
                    table_hbm.at[idx16],
                    rows_v.at[b, pl.ds(sub * 16, 16)], gsem[b]).start()

        def g_wait(i, b):
            for sub in range(_CH // 16):
                pltpu.make_async_copy(
                    table_hbm.at[idx_v[pl.ds(sub * 16, 16)]],
                    rows_v.at[b, pl.ds(sub * 16, 16)], gsem[b]).wait()

        def w_desc(i, b):
            return pltpu.make_async_copy(
                rows_v.at[b], out_hbm.at[pl.ds(base + i * _CH, _CH)], wsem[b])

        g_start(0, 0)
        g_start(1, 1)

        def body(j, carry):
            i0 = 2 * j
            i1 = 2 * j + 1
            g_wait(i0, 0)
            w_desc(i0, 0).start()
            w_desc(i0, 0).wait()
            g_start(i0 + 2, 0)
            g_wait(i1, 1)
            w_desc(i1, 1).start()
            w_desc(i1, 1).wait()
            g_start(i1 + 2, 1)
            return carry

        lax.fori_loop(0, _NCH // 2 - 1, body, 0)
        g_wait(_NCH - 2, 0)
        w_desc(_NCH - 2, 0).start()
        g_wait(_NCH - 1, 1)
        w_desc(_NCH - 1, 1).start()
        w_desc(_NCH - 2, 0).wait()
        w_desc(_NCH - 1, 1).wait()

    return _sc_gather


def _gather_call(kv, idx_flat):
    return _build_sc_gather()(kv, idx_flat)


_QT5 = 200


def _att_body(de_ref, qm_ref, kv_ref, mWo_ref, mbo_ref, W1_ref, b1_ref,
              g_ref, be_ref, W2_ref, b2_ref, y_ref):
    qm = qm_ref[...]
    u = lax.bitcast_convert_type(kv_ref[...], jnp.uint32)
    kg = lax.bitcast_convert_type(
        (u & 0xffff).astype(jnp.uint16), jnp.bfloat16).astype(jnp.float32)
    vg = lax.bitcast_convert_type(
        (u >> 16).astype(jnp.uint16), jnp.bfloat16).astype(jnp.float32)
    p = kg * qm[:, None, :]
    scs = [jnp.sum(p[:, :, h * _DH:(h + 1) * _DH], axis=-1) for h in range(_NH)]
    s = jnp.stack(scs, axis=-1) * (1.0 / math.sqrt(_DH))
    m = jnp.max(s, axis=1, keepdims=True)
    e = jnp.exp(s - m)
    a = e / jnp.sum(e, axis=1, keepdims=True)
    ab = jnp.concatenate(
        [jnp.broadcast_to(a[:, :, h:h + 1], (_QT5, _TOPK, _DH)) for h in range(_NH)],
        axis=-1)
    o = jnp.sum(ab * vg, axis=1)
    retr = _dot(o, mWo_ref[...]) + mbo_ref[...]
    comb = jnp.concatenate([de_ref[...], retr], axis=-1)
    z = jax.nn.relu(_ln(_dot(comb, W1_ref[...]) + b1_ref[...], g_ref[...], be_ref[...]))
    y_ref[...] = _dot(z, W2_ref[...]) + b2_ref[...]


def _att_call(de, qm, kvg, mWo, mbo, out_W1, out_b1, out_g, out_be, out_W2, out_b2):
    grid = (_R // _QT5,)

    def full(a):
        return pl.BlockSpec(a.shape, lambda i: (0,) * a.ndim)

    consts = [mWo, mbo, out_W1, out_b1, out_g, out_be, out_W2, out_b2]
    return pl.pallas_call(
        _att_body,
        grid=grid,
        in_specs=[pl.BlockSpec((_QT5, _RD), lambda i: (i, 0)),
                  pl.BlockSpec((_QT5, _RD), lambda i: (i, 0)),
                  pl.BlockSpec((_QT5, _TOPK, _RD), lambda i: (i, 0, 0))]
                 + [full(c) for c in consts],
        out_specs=pl.BlockSpec((_QT5, _HOR * _OD), lambda i: (i, 0)),
        out_shape=jax.ShapeDtypeStruct((_R, _HOR * _OD), jnp.float32),
    )(de, qm, kvg, *consts)


def kernel(history_data, W_temp, b_temp, spatial_emb, enc_W1, enc_b1, enc_g1,
           enc_be1, enc_W2, enc_b2, enc_g2, enc_be2, W_h2e, b_h2e, Wq, bq,
           Wk, bk, Wv, bv, mWq, mbq, mWk, mbk, mWv, mbv, mWo, mbo,
           out_W1, out_b1, out_g, out_be, out_W2, out_b2, store):
    x2 = history_data.transpose(0, 2, 1, 3).reshape(_R, _L * _C)
    spf = jnp.broadcast_to(spatial_emb[None], (_B, _N, _SD)).reshape(_R, _SD)

    r1 = lambda v: v.reshape(1, -1)
    r3 = lambda v: v.reshape(_EL, 1, -1)

    de, q, qm = _encoder_call(
        x2, spf, W_temp, r1(b_temp), enc_W1, r3(enc_b1), r3(enc_g1), r3(enc_be1),
        enc_W2, r3(enc_b2), r3(enc_g2), r3(enc_be2), W_h2e, r1(b_h2e),
        Wq, r1(bq), mWq, r1(mbq))

    kv = _kv_call(store, Wk, r1(bk), mWk, r1(mbk), Wv, r1(bv), mWv, r1(mbv))

    idx = _topk_call(q, store)

    idx_flat = jnp.concatenate(
        [idx.reshape(_BT), jnp.zeros((_BTP - _BT,), jnp.int32)])
    g = _gather_call(kv, idx_flat)[:_BT]

    y = _att_call(de, qm, g.reshape(_R, _TOPK, _RD), mWo, r1(mbo),
                  out_W1, r1(out_b1), r1(out_g), r1(out_be), out_W2, r1(out_b2))

    return y.reshape(_B, _N, _HOR, _OD).transpose(0, 2, 1, 3)


# speedup vs baseline: 4.1653x; 1.0579x over previous
import functools
import math

import jax
import jax.numpy as jnp
from jax import lax
from jax.experimental import pallas as pl
from jax.experimental.pallas import tpu as pltpu
from jax.experimental.pallas import tpu_sc as plsc

_B, _L, _N, _C = 32, 12, 325, 3
_HOR, _OD = 12, 1
_TD, _SD = 64, 32
_F = _TD + _SD
_ED, _RD = 512, 512
_NH = 8
_DH = _RD // _NH
_KDOCS = 4096
_TOPK = 8
_EL = 3
_EPS = 1e-5
_R = _B * _N

_HI = lax.Precision.HIGHEST


def _ln(x, g, b):
    m = x.mean(-1, keepdims=True)
    v = ((x - m) ** 2).mean(-1, keepdims=True)
    return (x - m) * lax.rsqrt(v + _EPS) * g + b


def _dot(a, b):
    return jnp.dot(a, b, preferred_element_type=jnp.float32)


_QT1 = 400


def _enc_body(x_ref, sp_ref, Wt_ref, bt_ref, W1_ref, b1_ref, g1_ref, be1_ref,
              W2_ref, b2_ref, g2_ref, be2_ref, Wh_ref, bh_ref, Wq_ref, bq_ref,
              mWq_ref, mbq_ref, de_ref, q_ref, qm_ref):
    t = _dot(x_ref[...], Wt_ref[...]) + bt_ref[...]
    h = jnp.concatenate([t, sp_ref[...]], axis=-1)
    for i in range(_EL):
        h = jax.nn.relu(_ln(_dot(h, W1_ref[i]) + b1_ref[i], g1_ref[i], be1_ref[i]))
        h = jax.nn.relu(_ln(_dot(h, W2_ref[i]) + b2_ref[i], g2_ref[i], be2_ref[i]))
    de = _dot(h, Wh_ref[...]) + bh_ref[...]
    q = _dot(de, Wq_ref[...]) + bq_ref[...]
    de_ref[...] = de
    q_ref[...] = q
    qm_ref[...] = _dot(q, mWq_ref[...]) + mbq_ref[...]


def _encoder_call(x2, spf, W_temp, b_temp, enc_W1, enc_b1, enc_g1, enc_be1,
                  enc_W2, enc_b2, enc_g2, enc_be2, W_h2e, b_h2e, Wq, bq, mWq, mbq):
    grid = (_R // _QT1,)
    row = pl.BlockSpec((_QT1, None), lambda i: (i, 0))

    def full(a):
        return pl.BlockSpec(a.shape, lambda i: (0,) * a.ndim)

    row_specs = [pl.BlockSpec((_QT1, x2.shape[1]), lambda i: (i, 0)),
                 pl.BlockSpec((_QT1, spf.shape[1]), lambda i: (i, 0))]
    consts = [W_temp, b_temp, enc_W1, enc_b1, enc_g1, enc_be1,
              enc_W2, enc_b2, enc_g2, enc_be2, W_h2e, b_h2e, Wq, bq, mWq, mbq]
    out_spec = pl.BlockSpec((_QT1, _RD), lambda i: (i, 0))
    return pl.pallas_call(
        _enc_body,
        grid=grid,
        in_specs=row_specs + [full(c) for c in consts],
        out_specs=[out_spec, out_spec, out_spec],
        out_shape=[jax.ShapeDtypeStruct((_R, _RD), jnp.float32)] * 3,
    )(x2, spf, *consts)


def _kv_body(st_ref, Wk_ref, bk_ref, mWk_ref, mbk_ref,
             Wv_ref, bv_ref, mWv_ref, mbv_ref, kv_ref):
    s = st_ref[...]
    kk = _dot(_dot(s, Wk_ref[...]) + bk_ref[...], mWk_ref[...]) + mbk_ref[...]
    vv = _dot(_dot(s, Wv_ref[...]) + bv_ref[...], mWv_ref[...]) + mbv_ref[...]
    k16 = lax.bitcast_convert_type(kk.astype(jnp.bfloat16), jnp.uint16).astype(jnp.uint32)
    v16 = lax.bitcast_convert_type(vv.astype(jnp.bfloat16), jnp.uint16).astype(jnp.uint32)
    kv_ref[...] = lax.bitcast_convert_type(k16 | (v16 << 16), jnp.int32)


def _kv_call(store, Wk, bk, mWk, mbk, Wv, bv, mWv, mbv):
    T = 512
    grid = (_KDOCS // T,)

    def full(a):
        return pl.BlockSpec(a.shape, lambda i: (0,) * a.ndim)

    consts = [Wk, bk, mWk, mbk, Wv, bv, mWv, mbv]
    return pl.pallas_call(
        _kv_body,
        grid=grid,
        in_specs=[pl.BlockSpec((T, _RD), lambda i: (i, 0))] + [full(c) for c in consts],
        out_specs=pl.BlockSpec((T, _RD), lambda i: (i, 0)),
        out_shape=jax.ShapeDtypeStruct((_KDOCS, _RD), jnp.int32),
    )(store, *consts)


_NS = 4
_RS = _R // _NS
_QT3 = 200


def _topk_body(q_ref, st_ref, idx_ref):
    s = lax.dot_general(q_ref[...], st_ref[...], (((1,), (1,)), ((), ())),
                        preferred_element_type=jnp.float32)
    cols = lax.broadcasted_iota(jnp.int32, (_QT3, _KDOCS), 1)
    outs = []
    for _ in range(_TOPK):
        m = jnp.max(s, axis=1, keepdims=True)
        cand = jnp.where(s >= m, cols, _KDOCS)
        cj = jnp.min(cand, axis=1, keepdims=True)
        outs.append(cj)
        s = jnp.where(cols == cj, -jnp.inf, s)
    idx_ref[...] = jnp.concatenate(outs, axis=1)


def _topk_call(q, store):
    rows = q.shape[0]
    grid = (rows // _QT3,)
    return pl.pallas_call(
        _topk_body,
        grid=grid,
        in_specs=[pl.BlockSpec((_QT3, _RD), lambda i: (i, 0)),
                  pl.BlockSpec((_KDOCS, _RD), lambda i: (0, 0))],
        out_specs=pl.BlockSpec((_QT3, _TOPK), lambda i: (i, 0)),
        out_shape=jax.ShapeDtypeStruct((rows, _TOPK), jnp.int32),
    )(q, store)


_NW = 32
_BT = _RS * _TOPK
_BTP = 21504
_BPW = _BTP // _NW
_CH = 16


_NCH = _BPW // _CH


@functools.cache
def _build_sc_gather():
    @functools.partial(
        pl.kernel,
        mesh=plsc.VectorSubcoreMesh(core_axis_name="c", subcore_axis_name="s"),
        out_type=jax.ShapeDtypeStruct((_BTP, _RD), jnp.int32),
        scratch_types=[
            pltpu.VMEM((_BPW,), jnp.int32),
            pltpu.VMEM((2, _CH, _RD), jnp.int32),
        ] + [pltpu.SemaphoreType.DMA] * 4,
    )
    def _sc_gather(table_hbm, idx_hbm, out_hbm, idx_v, rows_v, *sems):
        gsem = sems[:2]
        wsem = sems[2:]
        wid = lax.axis_index("s") * 2 + lax.axis_index("c")
        base = wid * _BPW
        pltpu.sync_copy(idx_hbm.at[pl.ds(base, _BPW)], idx_v)

        def g_start(i, b):
            for sub in range(_CH // 16):
                idx16 = idx_v[pl.ds(i * _CH + sub * 16, 16)]
                pltpu.make_async_copy(
                    table_hbm.at[idx16],
                    rows_v.at[b, pl.ds(sub * 16, 16)], gsem[b]).start()

        def g_wait(i, b):
            for sub in range(_CH // 16):
                pltpu.make_async_copy(
                    table_hbm.at[idx_v[pl.ds(sub * 16, 16)]],
                    rows_v.at[b, pl.ds(sub * 16, 16)], gsem[b]).wait()

        def w_desc(i, b):
            return pltpu.make_async_copy(
                rows_v.at[b], out_hbm.at[pl.ds(base + i * _CH, _CH)], wsem[b])

        g_start(0, 0)
        g_start(1, 1)

        def body(j, carry):
            i0 = 2 * j
            i1 = 2 * j + 1
            g_wait(i0, 0)
            w_desc(i0, 0).start()
            w_desc(i0, 0).wait()
            g_start(i0 + 2, 0)
            g_wait(i1, 1)
            w_desc(i1, 1).start()
            w_desc(i1, 1).wait()
            g_start(i1 + 2, 1)
            return carry

        lax.fori_loop(0, _NCH // 2 - 1, body, 0)
        g_wait(_NCH - 2, 0)
        w_desc(_NCH - 2, 0).start()
        g_wait(_NCH - 1, 1)
        w_desc(_NCH - 1, 1).start()
        w_desc(_NCH - 2, 0).wait()
        w_desc(_NCH - 1, 1).wait()

    return _sc_gather


def _gather_call(kv, idx_flat):
    return _build_sc_gather()(kv, idx_flat)


_QT5 = 200


def _att_body(de_ref, qm_ref, kv_ref, mWo_ref, mbo_ref, W1_ref, b1_ref,
              g_ref, be_ref, W2_ref, b2_ref, y_ref):
    qm = qm_ref[...]
    u = lax.bitcast_convert_type(kv_ref[...], jnp.uint32)
    kg = lax.bitcast_convert_type(
        (u & 0xffff).astype(jnp.uint16), jnp.bfloat16).astype(jnp.float32)
    vg = lax.bitcast_convert_type(
        (u >> 16).astype(jnp.uint16), jnp.bfloat16).astype(jnp.float32)
    p = kg * qm[:, None, :]
    scs = [jnp.sum(p[:, :, h * _DH:(h + 1) * _DH], axis=-1) for h in range(_NH)]
    s = jnp.stack(scs, axis=-1) * (1.0 / math.sqrt(_DH))
    m = jnp.max(s, axis=1, keepdims=True)
    e = jnp.exp(s - m)
    a = e / jnp.sum(e, axis=1, keepdims=True)
    ab = jnp.concatenate(
        [jnp.broadcast_to(a[:, :, h:h + 1], (_QT5, _TOPK, _DH)) for h in range(_NH)],
        axis=-1)
    o = jnp.sum(ab * vg, axis=1)
    retr = _dot(o, mWo_ref[...]) + mbo_ref[...]
    comb = jnp.concatenate([de_ref[...], retr], axis=-1)
    z = jax.nn.relu(_ln(_dot(comb, W1_ref[...]) + b1_ref[...], g_ref[...], be_ref[...]))
    y_ref[...] = _dot(z, W2_ref[...]) + b2_ref[...]


def _att_call(de, qm, kvg, mWo, mbo, out_W1, out_b1, out_g, out_be, out_W2, out_b2):
    rows = de.shape[0]
    grid = (rows // _QT5,)

    def full(a):
        return pl.BlockSpec(a.shape, lambda i: (0,) * a.ndim)

    consts = [mWo, mbo, out_W1, out_b1, out_g, out_be, out_W2, out_b2]
    return pl.pallas_call(
        _att_body,
        grid=grid,
        in_specs=[pl.BlockSpec((_QT5, _RD), lambda i: (i, 0)),
                  pl.BlockSpec((_QT5, _RD), lambda i: (i, 0)),
                  pl.BlockSpec((_QT5, _TOPK, _RD), lambda i: (i, 0, 0))]
                 + [full(c) for c in consts],
        out_specs=pl.BlockSpec((_QT5, _HOR * _OD), lambda i: (i, 0)),
        out_shape=jax.ShapeDtypeStruct((rows, _HOR * _OD), jnp.float32),
    )(de, qm, kvg, *consts)


def kernel(history_data, W_temp, b_temp, spatial_emb, enc_W1, enc_b1, enc_g1,
           enc_be1, enc_W2, enc_b2, enc_g2, enc_be2, W_h2e, b_h2e, Wq, bq,
           Wk, bk, Wv, bv, mWq, mbq, mWk, mbk, mWv, mbv, mWo, mbo,
           out_W1, out_b1, out_g, out_be, out_W2, out_b2, store):
    x2 = history_data.transpose(0, 2, 1, 3).reshape(_R, _L * _C)
    spf = jnp.broadcast_to(spatial_emb[None], (_B, _N, _SD)).reshape(_R, _SD)

    r1 = lambda v: v.reshape(1, -1)
    r3 = lambda v: v.reshape(_EL, 1, -1)

    de, q, qm = _encoder_call(
        x2, spf, W_temp, r1(b_temp), enc_W1, r3(enc_b1), r3(enc_g1), r3(enc_be1),
        enc_W2, r3(enc_b2), r3(enc_g2), r3(enc_be2), W_h2e, r1(b_h2e),
        Wq, r1(bq), mWq, r1(mbq))

    kv = _kv_call(store, Wk, r1(bk), mWk, r1(mbk), Wv, r1(bv), mWv, r1(mbv))

    pad = jnp.zeros((_BTP - _BT,), jnp.int32)
    ys = []
    for s in range(_NS):
        sl = slice(s * _RS, (s + 1) * _RS)
        idx = _topk_call(q[sl], store)
        g = _gather_call(kv, jnp.concatenate([idx.reshape(_BT), pad]))[:_BT]
        ys.append(_att_call(de[sl], qm[sl], g.reshape(_RS, _TOPK, _RD),
                            mWo, r1(mbo), out_W1, r1(out_b1), r1(out_g),
                            r1(out_be), out_W2, r1(out_b2)))
    y = jnp.concatenate(ys, axis=0)

    return y.reshape(_B, _N, _HOR, _OD).transpose(0, 2, 1, 3)


# speedup vs baseline: 4.1968x; 1.0076x over previous
import functools
import math

import jax
import jax.numpy as jnp
from jax import lax
from jax.experimental import pallas as pl
from jax.experimental.pallas import tpu as pltpu
from jax.experimental.pallas import tpu_sc as plsc

_B, _L, _N, _C = 32, 12, 325, 3
_HOR, _OD = 12, 1
_TD, _SD = 64, 32
_F = _TD + _SD
_ED, _RD = 512, 512
_NH = 8
_DH = _RD // _NH
_KDOCS = 4096
_TOPK = 8
_EL = 3
_EPS = 1e-5
_R = _B * _N

_HI = lax.Precision.HIGHEST


def _ln(x, g, b):
    m = x.mean(-1, keepdims=True)
    v = ((x - m) ** 2).mean(-1, keepdims=True)
    return (x - m) * lax.rsqrt(v + _EPS) * g + b


def _dot(a, b):
    return jnp.dot(a, b, preferred_element_type=jnp.float32)


_QT1 = 400


def _enc_body(x_ref, sp_ref, Wt_ref, bt_ref, W1_ref, b1_ref, g1_ref, be1_ref,
              W2_ref, b2_ref, g2_ref, be2_ref, Wh_ref, bh_ref, Wq_ref, bq_ref,
              mWq_ref, mbq_ref, de_ref, q_ref, qm_ref):
    t = _dot(x_ref[...], Wt_ref[...]) + bt_ref[...]
    h = jnp.concatenate([t, sp_ref[...]], axis=-1)
    for i in range(_EL):
        h = jax.nn.relu(_ln(_dot(h, W1_ref[i]) + b1_ref[i], g1_ref[i], be1_ref[i]))
        h = jax.nn.relu(_ln(_dot(h, W2_ref[i]) + b2_ref[i], g2_ref[i], be2_ref[i]))
    de = _dot(h, Wh_ref[...]) + bh_ref[...]
    q = _dot(de, Wq_ref[...]) + bq_ref[...]
    de_ref[...] = de
    q_ref[...] = q
    qm_ref[...] = _dot(q, mWq_ref[...]) + mbq_ref[...]


def _encoder_call(x2, spf, W_temp, b_temp, enc_W1, enc_b1, enc_g1, enc_be1,
                  enc_W2, enc_b2, enc_g2, enc_be2, W_h2e, b_h2e, Wq, bq, mWq, mbq):
    grid = (_R // _QT1,)
    row = pl.BlockSpec((_QT1, None), lambda i: (i, 0))

    def full(a):
        return pl.BlockSpec(a.shape, lambda i: (0,) * a.ndim)

    row_specs = [pl.BlockSpec((_QT1, x2.shape[1]), lambda i: (i, 0)),
                 pl.BlockSpec((_QT1, spf.shape[1]), lambda i: (i, 0))]
    consts = [W_temp, b_temp, enc_W1, enc_b1, enc_g1, enc_be1,
              enc_W2, enc_b2, enc_g2, enc_be2, W_h2e, b_h2e, Wq, bq, mWq, mbq]
    out_spec = pl.BlockSpec((_QT1, _RD), lambda i: (i, 0))
    return pl.pallas_call(
        _enc_body,
        grid=grid,
        in_specs=row_specs + [full(c) for c in consts],
        out_specs=[out_spec, out_spec, out_spec],
        out_shape=[jax.ShapeDtypeStruct((_R, _RD), jnp.float32)] * 3,
    )(x2, spf, *consts)


def _kv_body(st_ref, Wk_ref, bk_ref, mWk_ref, mbk_ref,
             Wv_ref, bv_ref, mWv_ref, mbv_ref, kv_ref):
    s = st_ref[...]
    kk = _dot(_dot(s, Wk_ref[...]) + bk_ref[...], mWk_ref[...]) + mbk_ref[...]
    vv = _dot(_dot(s, Wv_ref[...]) + bv_ref[...], mWv_ref[...]) + mbv_ref[...]
    k16 = lax.bitcast_convert_type(kk.astype(jnp.bfloat16), jnp.uint16).astype(jnp.uint32)
    v16 = lax.bitcast_convert_type(vv.astype(jnp.bfloat16), jnp.uint16).astype(jnp.uint32)
    kv_ref[...] = lax.bitcast_convert_type(k16 | (v16 << 16), jnp.int32)


def _kv_call(store, Wk, bk, mWk, mbk, Wv, bv, mWv, mbv):
    T = 512
    grid = (_KDOCS // T,)

    def full(a):
        return pl.BlockSpec(a.shape, lambda i: (0,) * a.ndim)

    consts = [Wk, bk, mWk, mbk, Wv, bv, mWv, mbv]
    return pl.pallas_call(
        _kv_body,
        grid=grid,
        in_specs=[pl.BlockSpec((T, _RD), lambda i: (i, 0))] + [full(c) for c in consts],
        out_specs=pl.BlockSpec((T, _RD), lambda i: (i, 0)),
        out_shape=jax.ShapeDtypeStruct((_KDOCS, _RD), jnp.int32),
    )(store, *consts)


_NS = 4
_RS = _R // _NS
_QT3 = 200


def _topk_body(q_ref, st_ref, idx_ref):
    s = lax.dot_general(q_ref[...], st_ref[...], (((1,), (1,)), ((), ())),
                        preferred_element_type=jnp.float32)
    cols = lax.broadcasted_iota(jnp.int32, (_QT3, _KDOCS), 1)
    outs = []
    for _ in range(_TOPK):
        m = jnp.max(s, axis=1, keepdims=True)
        cand = jnp.where(s >= m, cols, _KDOCS)
        cj = jnp.min(cand, axis=1, keepdims=True)
        outs.append(cj)
        s = jnp.where(cols == cj, -jnp.inf, s)
    idx_ref[...] = jnp.concatenate(outs, axis=1)


def _topk_call(q, store):
    rows = q.shape[0]
    grid = (rows // _QT3,)
    return pl.pallas_call(
        _topk_body,
        grid=grid,
        in_specs=[pl.BlockSpec((_QT3, _RD), lambda i: (i, 0)),
                  pl.BlockSpec((_KDOCS, _RD), lambda i: (0, 0))],
        out_specs=pl.BlockSpec((_QT3, _TOPK), lambda i: (i, 0)),
        out_shape=jax.ShapeDtypeStruct((rows, _TOPK), jnp.int32),
    )(q, store)


_NW = 32
_BT = _RS * _TOPK
_BTP = 21504
_BPW = _BTP // _NW
_CH = 48


_NCH = _BPW // _CH


@functools.cache
def _build_sc_gather():
    @functools.partial(
        pl.kernel,
        mesh=plsc.VectorSubcoreMesh(core_axis_name="c", subcore_axis_name="s"),
        out_type=jax.ShapeDtypeStruct((_BTP, _RD), jnp.int32),
        scratch_types=[
            pltpu.VMEM((_BPW,), jnp.int32),
            pltpu.VMEM((2, _CH, _RD), jnp.int32),
        ] + [pltpu.SemaphoreType.DMA] * 4,
    )
    def _sc_gather(table_hbm, idx_hbm, out_hbm, idx_v, rows_v, *sems):
        gsem = sems[:2]
        wsem = sems[2:]
        wid = lax.axis_index("s") * 2 + lax.axis_index("c")
        base = wid * _BPW
        pltpu.sync_copy(idx_hbm.at[pl.ds(base, _BPW)], idx_v)

        def g_start(i, b):
            for sub in range(_CH // 16):
                idx16 = idx_v[pl.ds(i * _CH + sub * 16, 16)]
                pltpu.make_async_copy(
                    table_hbm.at[idx16],
                    rows_v.at[b, pl.ds(sub * 16, 16)], gsem[b]).start()

        def g_wait(i, b):
            for sub in range(_CH // 16):
                pltpu.make_async_copy(
                    table_hbm.at[idx_v[pl.ds(sub * 16, 16)]],
                    rows_v.at[b, pl.ds(sub * 16, 16)], gsem[b]).wait()

        def w_desc(i, b):
            return pltpu.make_async_copy(
                rows_v.at[b], out_hbm.at[pl.ds(base + i * _CH, _CH)], wsem[b])

        g_start(0, 0)
        g_start(1, 1)

        def body(j, carry):
            i0 = 2 * j
            i1 = 2 * j + 1
            g_wait(i0, 0)
            w_desc(i0, 0).start()
            w_desc(i0, 0).wait()
            g_start(i0 + 2, 0)
            g_wait(i1, 1)
            w_desc(i1, 1).start()
            w_desc(i1, 1).wait()
            g_start(i1 + 2, 1)
            return carry

        lax.fori_loop(0, _NCH // 2 - 1, body, 0)
        g_wait(_NCH - 2, 0)
        w_desc(_NCH - 2, 0).start()
        g_wait(_NCH - 1, 1)
        w_desc(_NCH - 1, 1).start()
        w_desc(_NCH - 2, 0).wait()
        w_desc(_NCH - 1, 1).wait()

    return _sc_gather


def _gather_call(kv, idx_flat):
    return _build_sc_gather()(kv, idx_flat)


_QT5 = 200


def _att_body(de_ref, qm_ref, kv_ref, mWo_ref, mbo_ref, W1_ref, b1_ref,
              g_ref, be_ref, W2_ref, b2_ref, y_ref):
    qm = qm_ref[...]
    u = lax.bitcast_convert_type(kv_ref[...], jnp.uint32)
    kg = lax.bitcast_convert_type(
        (u & 0xffff).astype(jnp.uint16), jnp.bfloat16).astype(jnp.float32)
    vg = lax.bitcast_convert_type(
        (u >> 16).astype(jnp.uint16), jnp.bfloat16).astype(jnp.float32)
    p = kg * qm[:, None, :]
    scs = [jnp.sum(p[:, :, h * _DH:(h + 1) * _DH], axis=-1) for h in range(_NH)]
    s = jnp.stack(scs, axis=-1) * (1.0 / math.sqrt(_DH))
    m = jnp.max(s, axis=1, keepdims=True)
    e = jnp.exp(s - m)
    a = e / jnp.sum(e, axis=1, keepdims=True)
    ab = jnp.concatenate(
        [jnp.broadcast_to(a[:, :, h:h + 1], (_QT5, _TOPK, _DH)) for h in range(_NH)],
        axis=-1)
    o = jnp.sum(ab * vg, axis=1)
    retr = _dot(o, mWo_ref[...]) + mbo_ref[...]
    comb = jnp.concatenate([de_ref[...], retr], axis=-1)
    z = jax.nn.relu(_ln(_dot(comb, W1_ref[...]) + b1_ref[...], g_ref[...], be_ref[...]))
    y_ref[...] = _dot(z, W2_ref[...]) + b2_ref[...]


def _att_call(de, qm, kvg, mWo, mbo, out_W1, out_b1, out_g, out_be, out_W2, out_b2):
    rows = de.shape[0]
    grid = (rows // _QT5,)

    def full(a):
        return pl.BlockSpec(a.shape, lambda i: (0,) * a.ndim)

    consts = [mWo, mbo, out_W1, out_b1, out_g, out_be, out_W2, out_b2]
    return pl.pallas_call(
        _att_body,
        grid=grid,
        in_specs=[pl.BlockSpec((_QT5, _RD), lambda i: (i, 0)),
                  pl.BlockSpec((_QT5, _RD), lambda i: (i, 0)),
                  pl.BlockSpec((_QT5, _TOPK, _RD), lambda i: (i, 0, 0))]
                 + [full(c) for c in consts],
        out_specs=pl.BlockSpec((_QT5, _HOR * _OD), lambda i: (i, 0)),
        out_shape=jax.ShapeDtypeStruct((rows, _HOR * _OD), jnp.float32),
    )(de, qm, kvg, *consts)


def kernel(history_data, W_temp, b_temp, spatial_emb, enc_W1, enc_b1, enc_g1,
           enc_be1, enc_W2, enc_b2, enc_g2, enc_be2, W_h2e, b_h2e, Wq, bq,
           Wk, bk, Wv, bv, mWq, mbq, mWk, mbk, mWv, mbv, mWo, mbo,
           out_W1, out_b1, out_g, out_be, out_W2, out_b2, store):
    x2 = history_data.transpose(0, 2, 1, 3).reshape(_R, _L * _C)
    spf = jnp.broadcast_to(spatial_emb[None], (_B, _N, _SD)).reshape(_R, _SD)

    r1 = lambda v: v.reshape(1, -1)
    r3 = lambda v: v.reshape(_EL, 1, -1)

    de, q, qm = _encoder_call(
        x2, spf, W_temp, r1(b_temp), enc_W1, r3(enc_b1), r3(enc_g1), r3(enc_be1),
        enc_W2, r3(enc_b2), r3(enc_g2), r3(enc_be2), W_h2e, r1(b_h2e),
        Wq, r1(bq), mWq, r1(mbq))

    kv = _kv_call(store, Wk, r1(bk), mWk, r1(mbk), Wv, r1(bv), mWv, r1(mbv))

    pad = jnp.zeros((_BTP - _BT,), jnp.int32)
    ys = []
    for s in range(_NS):
        sl = slice(s * _RS, (s + 1) * _RS)
        idx = _topk_call(q[sl], store)
        g = _gather_call(kv, jnp.concatenate([idx.reshape(_BT), pad]))[:_BT]
        ys.append(_att_call(de[sl], qm[sl], g.reshape(_RS, _TOPK, _RD),
                            mWo, r1(mbo), out_W1, r1(out_b1), r1(out_g),
                            r1(out_be), out_W2, r1(out_b2)))
    y = jnp.concatenate(ys, axis=0)

    return y.reshape(_B, _N, _HOR, _OD).transpose(0, 2, 1, 3)


# speedup vs baseline: 4.3395x; 1.0340x over previous
import functools
import math

import jax
import jax.numpy as jnp
from jax import lax
from jax.experimental import pallas as pl
from jax.experimental.pallas import tpu as pltpu
from jax.experimental.pallas import tpu_sc as plsc

_B, _L, _N, _C = 32, 12, 325, 3
_HOR, _OD = 12, 1
_TD, _SD = 64, 32
_F = _TD + _SD
_ED, _RD = 512, 512
_NH = 8
_DH = _RD // _NH
_KDOCS = 4096
_TOPK = 8
_EL = 3
_EPS = 1e-5
_R = _B * _N

_HI = lax.Precision.HIGHEST


def _ln(x, g, b):
    m = x.mean(-1, keepdims=True)
    v = ((x - m) ** 2).mean(-1, keepdims=True)
    return (x - m) * lax.rsqrt(v + _EPS) * g + b


def _dot(a, b):
    return jnp.dot(a, b, preferred_element_type=jnp.float32)


_QT1 = 400


def _enc_body(x_ref, sp_ref, Wt_ref, bt_ref, W1_ref, b1_ref, g1_ref, be1_ref,
              W2_ref, b2_ref, g2_ref, be2_ref, Wh_ref, bh_ref, Wq_ref, bq_ref,
              mWq_ref, mbq_ref, de_ref, q_ref, qm_ref):
    t = _dot(x_ref[...], Wt_ref[...]) + bt_ref[...]
    h = jnp.concatenate([t, sp_ref[...]], axis=-1)
    for i in range(_EL):
        h = jax.nn.relu(_ln(_dot(h, W1_ref[i]) + b1_ref[i], g1_ref[i], be1_ref[i]))
        h = jax.nn.relu(_ln(_dot(h, W2_ref[i]) + b2_ref[i], g2_ref[i], be2_ref[i]))
    de = _dot(h, Wh_ref[...]) + bh_ref[...]
    q = _dot(de, Wq_ref[...]) + bq_ref[...]
    de_ref[...] = de
    q_ref[...] = q
    qm_ref[...] = _dot(q, mWq_ref[...]) + mbq_ref[...]


def _encoder_call(x2, spf, W_temp, b_temp, enc_W1, enc_b1, enc_g1, enc_be1,
                  enc_W2, enc_b2, enc_g2, enc_be2, W_h2e, b_h2e, Wq, bq, mWq, mbq):
    grid = (_R // _QT1,)
    row = pl.BlockSpec((_QT1, None), lambda i: (i, 0))

    def full(a):
        return pl.BlockSpec(a.shape, lambda i: (0,) * a.ndim)

    row_specs = [pl.BlockSpec((_QT1, x2.shape[1]), lambda i: (i, 0)),
                 pl.BlockSpec((_QT1, spf.shape[1]), lambda i: (i, 0))]
    consts = [W_temp, b_temp, enc_W1, enc_b1, enc_g1, enc_be1,
              enc_W2, enc_b2, enc_g2, enc_be2, W_h2e, b_h2e, Wq, bq, mWq, mbq]
    out_spec = pl.BlockSpec((_QT1, _RD), lambda i: (i, 0))
    return pl.pallas_call(
        _enc_body,
        grid=grid,
        in_specs=row_specs + [full(c) for c in consts],
        out_specs=[out_spec, out_spec, out_spec],
        out_shape=[jax.ShapeDtypeStruct((_R, _RD), jnp.float32)] * 3,
    )(x2, spf, *consts)


def _kv_body(st_ref, Wk_ref, bk_ref, mWk_ref, mbk_ref,
             Wv_ref, bv_ref, mWv_ref, mbv_ref, kv_ref):
    s = st_ref[...]
    kk = _dot(_dot(s, Wk_ref[...]) + bk_ref[...], mWk_ref[...]) + mbk_ref[...]
    vv = _dot(_dot(s, Wv_ref[...]) + bv_ref[...], mWv_ref[...]) + mbv_ref[...]
    k16 = lax.bitcast_convert_type(kk.astype(jnp.bfloat16), jnp.uint16).astype(jnp.uint32)
    v16 = lax.bitcast_convert_type(vv.astype(jnp.bfloat16), jnp.uint16).astype(jnp.uint32)
    kv_ref[...] = lax.bitcast_convert_type(k16 | (v16 << 16), jnp.int32)


def _kv_call(store, Wk, bk, mWk, mbk, Wv, bv, mWv, mbv):
    T = 512
    grid = (_KDOCS // T,)

    def full(a):
        return pl.BlockSpec(a.shape, lambda i: (0,) * a.ndim)

    consts = [Wk, bk, mWk, mbk, Wv, bv, mWv, mbv]
    return pl.pallas_call(
        _kv_body,
        grid=grid,
        in_specs=[pl.BlockSpec((T, _RD), lambda i: (i, 0))] + [full(c) for c in consts],
        out_specs=pl.BlockSpec((T, _RD), lambda i: (i, 0)),
        out_shape=jax.ShapeDtypeStruct((_KDOCS, _RD), jnp.int32),
    )(store, *consts)


_NS = 4
_RS = _R // _NS
_QT3 = 200


def _topk_body(q_ref, st_ref, idx_ref):
    s = lax.dot_general(q_ref[...], st_ref[...], (((1,), (1,)), ((), ())),
                        preferred_element_type=jnp.float32)
    cols = lax.broadcasted_iota(jnp.int32, (_QT3, _KDOCS), 1)
    outs = []
    for _ in range(_TOPK):
        m = jnp.max(s, axis=1, keepdims=True)
        hit = s == m
        cj = jnp.min(jnp.where(hit, cols, _KDOCS), axis=1, keepdims=True)
        outs.append(cj)
        s = jnp.where(hit, -jnp.inf, s)
    idx_ref[...] = jnp.concatenate(outs, axis=1)


def _topk_call(q, store):
    rows = q.shape[0]
    grid = (rows // _QT3,)
    return pl.pallas_call(
        _topk_body,
        grid=grid,
        in_specs=[pl.BlockSpec((_QT3, _RD), lambda i: (i, 0)),
                  pl.BlockSpec((_KDOCS, _RD), lambda i: (0, 0))],
        out_specs=pl.BlockSpec((_QT3, _TOPK), lambda i: (i, 0)),
        out_shape=jax.ShapeDtypeStruct((rows, _TOPK), jnp.int32),
    )(q, store)


_NW = 32
_BT = _RS * _TOPK
_BTP = 21504
_BPW = _BTP // _NW
_CH = 48


_NCH = _BPW // _CH


@functools.cache
def _build_sc_gather():
    @functools.partial(
        pl.kernel,
        mesh=plsc.VectorSubcoreMesh(core_axis_name="c", subcore_axis_name="s"),
        out_type=jax.ShapeDtypeStruct((_BTP, _RD), jnp.int32),
        scratch_types=[
            pltpu.VMEM((_BPW,), jnp.int32),
            pltpu.VMEM((2, _CH, _RD), jnp.int32),
        ] + [pltpu.SemaphoreType.DMA] * 4,
    )
    def _sc_gather(table_hbm, idx_hbm, out_hbm, idx_v, rows_v, *sems):
        gsem = sems[:2]
        wsem = sems[2:]
        wid = lax.axis_index("s") * 2 + lax.axis_index("c")
        base = wid * _BPW
        pltpu.sync_copy(idx_hbm.at[pl.ds(base, _BPW)], idx_v)

        def g_start(i, b):
            for sub in range(_CH // 16):
                idx16 = idx_v[pl.ds(i * _CH + sub * 16, 16)]
                pltpu.make_async_copy(
                    table_hbm.at[idx16],
                    rows_v.at[b, pl.ds(sub * 16, 16)], gsem[b]).start()

        def g_wait(i, b):
            for sub in range(_CH // 16):
                pltpu.make_async_copy(
                    table_hbm.at[idx_v[pl.ds(sub * 16, 16)]],
                    rows_v.at[b, pl.ds(sub * 16, 16)], gsem[b]).wait()

        def w_desc(i, b):
            return pltpu.make_async_copy(
                rows_v.at[b], out_hbm.at[pl.ds(base + i * _CH, _CH)], wsem[b])

        g_start(0, 0)
        g_start(1, 1)

        def body(j, carry):
            i0 = 2 * j
            i1 = 2 * j + 1
            g_wait(i0, 0)
            w_desc(i0, 0).start()
            w_desc(i0, 0).wait()
            g_start(i0 + 2, 0)
            g_wait(i1, 1)
            w_desc(i1, 1).start()
            w_desc(i1, 1).wait()
            g_start(i1 + 2, 1)
            return carry

        lax.fori_loop(0, _NCH // 2 - 1, body, 0)
        g_wait(_NCH - 2, 0)
        w_desc(_NCH - 2, 0).start()
        g_wait(_NCH - 1, 1)
        w_desc(_NCH - 1, 1).start()
        w_desc(_NCH - 2, 0).wait()
        w_desc(_NCH - 1, 1).wait()

    return _sc_gather


def _gather_call(kv, idx_flat):
    return _build_sc_gather()(kv, idx_flat)


_QT5 = 200


def _att_body(de_ref, qm_ref, kv_ref, mWo_ref, mbo_ref, W1_ref, b1_ref,
              g_ref, be_ref, W2_ref, b2_ref, y_ref):
    qm = qm_ref[...]
    u = lax.bitcast_convert_type(kv_ref[...], jnp.uint32)
    kg = lax.bitcast_convert_type(
        (u & 0xffff).astype(jnp.uint16), jnp.bfloat16).astype(jnp.float32)
    vg = lax.bitcast_convert_type(
        (u >> 16).astype(jnp.uint16), jnp.bfloat16).astype(jnp.float32)
    p = kg * qm[:, None, :]
    he = lax.broadcasted_iota(jnp.int32, (_NH, _RD), 0)
    de_i = lax.broadcasted_iota(jnp.int32, (_NH, _RD), 1) // _DH
    E = (he == de_i).astype(jnp.float32)
    ET = (lax.broadcasted_iota(jnp.int32, (_RD, _NH), 0) // _DH
          == lax.broadcasted_iota(jnp.int32, (_RD, _NH), 1)).astype(jnp.float32)
    s = _dot(p.reshape(_QT5 * _TOPK, _RD), ET).reshape(_QT5, _TOPK, _NH)
    s = s * (1.0 / math.sqrt(_DH))
    m = jnp.max(s, axis=1, keepdims=True)
    e = jnp.exp(s - m)
    a = e / jnp.sum(e, axis=1, keepdims=True)
    ab = _dot(a.reshape(_QT5 * _TOPK, _NH), E).reshape(_QT5, _TOPK, _RD)
    o = jnp.sum(ab * vg, axis=1)
    retr = _dot(o, mWo_ref[...]) + mbo_ref[...]
    comb = jnp.concatenate([de_ref[...], retr], axis=-1)
    z = jax.nn.relu(_ln(_dot(comb, W1_ref[...]) + b1_ref[...], g_ref[...], be_ref[...]))
    y_ref[...] = _dot(z, W2_ref[...]) + b2_ref[...]


def _att_call(de, qm, kvg, mWo, mbo, out_W1, out_b1, out_g, out_be, out_W2, out_b2):
    rows = de.shape[0]
    grid = (rows // _QT5,)

    def full(a):
        return pl.BlockSpec(a.shape, lambda i: (0,) * a.ndim)

    consts = [mWo, mbo, out_W1, out_b1, out_g, out_be, out_W2, out_b2]
    return pl.pallas_call(
        _att_body,
        grid=grid,
        in_specs=[pl.BlockSpec((_QT5, _RD), lambda i: (i, 0)),
                  pl.BlockSpec((_QT5, _RD), lambda i: (i, 0)),
                  pl.BlockSpec((_QT5, _TOPK, _RD), lambda i: (i, 0, 0))]
                 + [full(c) for c in consts],
        out_specs=pl.BlockSpec((_QT5, _HOR * _OD), lambda i: (i, 0)),
        out_shape=jax.ShapeDtypeStruct((rows, _HOR * _OD), jnp.float32),
    )(de, qm, kvg, *consts)


def kernel(history_data, W_temp, b_temp, spatial_emb, enc_W1, enc_b1, enc_g1,
           enc_be1, enc_W2, enc_b2, enc_g2, enc_be2, W_h2e, b_h2e, Wq, bq,
           Wk, bk, Wv, bv, mWq, mbq, mWk, mbk, mWv, mbv, mWo, mbo,
           out_W1, out_b1, out_g, out_be, out_W2, out_b2, store):
    x2 = history_data.transpose(0, 2, 1, 3).reshape(_R, _L * _C)
    spf = jnp.broadcast_to(spatial_emb[None], (_B, _N, _SD)).reshape(_R, _SD)

    r1 = lambda v: v.reshape(1, -1)
    r3 = lambda v: v.reshape(_EL, 1, -1)

    de, q, qm = _encoder_call(
        x2, spf, W_temp, r1(b_temp), enc_W1, r3(enc_b1), r3(enc_g1), r3(enc_be1),
        enc_W2, r3(enc_b2), r3(enc_g2), r3(enc_be2), W_h2e, r1(b_h2e),
        Wq, r1(bq), mWq, r1(mbq))

    kv = _kv_call(store, Wk, r1(bk), mWk, r1(mbk), Wv, r1(bv), mWv, r1(mbv))

    pad = jnp.zeros((_BTP - _BT,), jnp.int32)
    ys = []
    for s in range(_NS):
        sl = slice(s * _RS, (s + 1) * _RS)
        idx = _topk_call(q[sl], store)
        g = _gather_call(kv, jnp.concatenate([idx.reshape(_BT), pad]))[:_BT]
        ys.append(_att_call(de[sl], qm[sl], g.reshape(_RS, _TOPK, _RD),
                            mWo, r1(mbo), out_W1, r1(out_b1), r1(out_g),
                            r1(out_be), out_W2, r1(out_b2)))
    y = jnp.concatenate(ys, axis=0)

    return y.reshape(_B, _N, _HOR, _OD).transpose(0, 2, 1, 3)


# speedup vs baseline: 5.0363x; 1.1606x over previous
import functools
import math

import jax
import jax.numpy as jnp
from jax import lax
from jax.experimental import pallas as pl
from jax.experimental.pallas import tpu as pltpu
from jax.experimental.pallas import tpu_sc as plsc

_B, _L, _N, _C = 32, 12, 325, 3
_HOR, _OD = 12, 1
_TD, _SD = 64, 32
_F = _TD + _SD
_ED, _RD = 512, 512
_NH = 8
_DH = _RD // _NH
_KDOCS = 4096
_TOPK = 8
_EL = 3
_EPS = 1e-5
_R = _B * _N

_HI = lax.Precision.HIGHEST


def _ln(x, g, b):
    m = x.mean(-1, keepdims=True)
    v = ((x - m) ** 2).mean(-1, keepdims=True)
    return (x - m) * lax.rsqrt(v + _EPS) * g + b


def _dot(a, b):
    return jnp.dot(a, b, preferred_element_type=jnp.float32)


_QT1 = 400


def _enc_body(x_ref, sp_ref, Wt_ref, bt_ref, W1_ref, b1_ref, g1_ref, be1_ref,
              W2_ref, b2_ref, g2_ref, be2_ref, Wh_ref, bh_ref, Wq_ref, bq_ref,
              mWq_ref, mbq_ref, de_ref, q_ref, qm_ref):
    t = _dot(x_ref[...], Wt_ref[...]) + bt_ref[...]
    h = jnp.concatenate([t, sp_ref[...]], axis=-1)
    for i in range(_EL):
        h = jax.nn.relu(_ln(_dot(h, W1_ref[i]) + b1_ref[i], g1_ref[i], be1_ref[i]))
        h = jax.nn.relu(_ln(_dot(h, W2_ref[i]) + b2_ref[i], g2_ref[i], be2_ref[i]))
    de = _dot(h, Wh_ref[...]) + bh_ref[...]
    q = _dot(de, Wq_ref[...]) + bq_ref[...]
    de_ref[...] = de
    q_ref[...] = q
    qm_ref[...] = _dot(q, mWq_ref[...]) + mbq_ref[...]


def _encoder_call(x2, spf, W_temp, b_temp, enc_W1, enc_b1, enc_g1, enc_be1,
                  enc_W2, enc_b2, enc_g2, enc_be2, W_h2e, b_h2e, Wq, bq, mWq, mbq):
    grid = (_R // _QT1,)
    row = pl.BlockSpec((_QT1, None), lambda i: (i, 0))

    def full(a):
        return pl.BlockSpec(a.shape, lambda i: (0,) * a.ndim)

    row_specs = [pl.BlockSpec((_QT1, x2.shape[1]), lambda i: (i, 0)),
                 pl.BlockSpec((_QT1, spf.shape[1]), lambda i: (i, 0))]
    consts = [W_temp, b_temp, enc_W1, enc_b1, enc_g1, enc_be1,
              enc_W2, enc_b2, enc_g2, enc_be2, W_h2e, b_h2e, Wq, bq, mWq, mbq]
    out_spec = pl.BlockSpec((_QT1, _RD), lambda i: (i, 0))
    return pl.pallas_call(
        _enc_body,
        grid=grid,
        in_specs=row_specs + [full(c) for c in consts],
        out_specs=[out_spec, out_spec, out_spec],
        out_shape=[jax.ShapeDtypeStruct((_R, _RD), jnp.float32)] * 3,
    )(x2, spf, *consts)


def _kv_body(st_ref, Wk_ref, bk_ref, mWk_ref, mbk_ref,
             Wv_ref, bv_ref, mWv_ref, mbv_ref, kv_ref):
    s = st_ref[...]
    kk = _dot(_dot(s, Wk_ref[...]) + bk_ref[...], mWk_ref[...]) + mbk_ref[...]
    vv = _dot(_dot(s, Wv_ref[...]) + bv_ref[...], mWv_ref[...]) + mbv_ref[...]
    k16 = lax.bitcast_convert_type(kk.astype(jnp.bfloat16), jnp.uint16).astype(jnp.uint32)
    v16 = lax.bitcast_convert_type(vv.astype(jnp.bfloat16), jnp.uint16).astype(jnp.uint32)
    kv_ref[...] = lax.bitcast_convert_type(k16 | (v16 << 16), jnp.int32)


def _kv_call(store, Wk, bk, mWk, mbk, Wv, bv, mWv, mbv):
    T = 512
    grid = (_KDOCS // T,)

    def full(a):
        return pl.BlockSpec(a.shape, lambda i: (0,) * a.ndim)

    consts = [Wk, bk, mWk, mbk, Wv, bv, mWv, mbv]
    return pl.pallas_call(
        _kv_body,
        grid=grid,
        in_specs=[pl.BlockSpec((T, _RD), lambda i: (i, 0))] + [full(c) for c in consts],
        out_specs=pl.BlockSpec((T, _RD), lambda i: (i, 0)),
        out_shape=jax.ShapeDtypeStruct((_KDOCS, _RD), jnp.int32),
    )(store, *consts)


_NS = 2
_RS = _R // _NS
_QT3 = 200


def _topk_body(q_ref, st_ref, idx_ref):
    s = lax.dot_general(q_ref[...], st_ref[...], (((1,), (1,)), ((), ())),
                        preferred_element_type=jnp.float32)
    cols = lax.broadcasted_iota(jnp.int32, (_QT3, _KDOCS), 1)
    outs = []
    for _ in range(_TOPK):
        m = jnp.max(s, axis=1, keepdims=True)
        hit = s == m
        cj = jnp.min(jnp.where(hit, cols, _KDOCS), axis=1, keepdims=True)
        outs.append(cj)
        s = jnp.where(hit, -jnp.inf, s)
    idx_ref[...] = jnp.concatenate(outs, axis=1)


def _topk_call(q, store):
    rows = q.shape[0]
    grid = (rows // _QT3,)
    return pl.pallas_call(
        _topk_body,
        grid=grid,
        in_specs=[pl.BlockSpec((_QT3, _RD), lambda i: (i, 0)),
                  pl.BlockSpec((_KDOCS, _RD), lambda i: (0, 0))],
        out_specs=pl.BlockSpec((_QT3, _TOPK), lambda i: (i, 0)),
        out_shape=jax.ShapeDtypeStruct((rows, _TOPK), jnp.int32),
    )(q, store)


_NW = 32
_BT = _RS * _TOPK
_BTP = 43008
_BPW = _BTP // _NW
_CH = 96


_NCH = _BPW // _CH


@functools.cache
def _build_sc_gather():
    @functools.partial(
        pl.kernel,
        mesh=plsc.VectorSubcoreMesh(core_axis_name="c", subcore_axis_name="s"),
        out_type=jax.ShapeDtypeStruct((_BTP, _RD), jnp.int32),
        scratch_types=[
            pltpu.VMEM((_BPW,), jnp.int32),
            pltpu.VMEM((2, _CH, _RD), jnp.int32),
        ] + [pltpu.SemaphoreType.DMA] * 4,
    )
    def _sc_gather(table_hbm, idx_hbm, out_hbm, idx_v, rows_v, *sems):
        gsem = sems[:2]
        wsem = sems[2:]
        wid = lax.axis_index("s") * 2 + lax.axis_index("c")
        base = wid * _BPW
        pltpu.sync_copy(idx_hbm.at[pl.ds(base, _BPW)], idx_v)

        def g_start(i, b):
            for sub in range(_CH // 16):
                idx16 = idx_v[pl.ds(i * _CH + sub * 16, 16)]
                pltpu.make_async_copy(
                    table_hbm.at[idx16],
                    rows_v.at[b, pl.ds(sub * 16, 16)], gsem[b]).start()

        def g_wait(i, b):
            for sub in range(_CH // 16):
                pltpu.make_async_copy(
                    table_hbm.at[idx_v[pl.ds(sub * 16, 16)]],
                    rows_v.at[b, pl.ds(sub * 16, 16)], gsem[b]).wait()

        def w_desc(i, b):
            return pltpu.make_async_copy(
                rows_v.at[b], out_hbm.at[pl.ds(base + i * _CH, _CH)], wsem[b])

        g_start(0, 0)
        g_start(1, 1)

        def body(j, carry):
            i0 = 2 * j
            i1 = 2 * j + 1
            g_wait(i0, 0)
            w_desc(i0, 0).start()
            w_desc(i0, 0).wait()
            g_start(i0 + 2, 0)
            g_wait(i1, 1)
            w_desc(i1, 1).start()
            w_desc(i1, 1).wait()
            g_start(i1 + 2, 1)
            return carry

        lax.fori_loop(0, _NCH // 2 - 1, body, 0)
        g_wait(_NCH - 2, 0)
        w_desc(_NCH - 2, 0).start()
        g_wait(_NCH - 1, 1)
        w_desc(_NCH - 1, 1).start()
        w_desc(_NCH - 2, 0).wait()
        w_desc(_NCH - 1, 1).wait()

    return _sc_gather


def _gather_call(kv, idx_flat):
    return _build_sc_gather()(kv, idx_flat)


_QT5 = 200


def _att_body(de_ref, qm_ref, kv_ref, mWo_ref, mbo_ref, W1_ref, b1_ref,
              g_ref, be_ref, W2_ref, b2_ref, y_ref):
    qm = qm_ref[...]
    u = lax.bitcast_convert_type(kv_ref[...], jnp.uint32)
    kg = lax.bitcast_convert_type(
        (u & 0xffff).astype(jnp.uint16), jnp.bfloat16).astype(jnp.float32)
    vg = lax.bitcast_convert_type(
        (u >> 16).astype(jnp.uint16), jnp.bfloat16).astype(jnp.float32)
    p = kg * qm[:, None, :]
    he = lax.broadcasted_iota(jnp.int32, (_NH, _RD), 0)
    de_i = lax.broadcasted_iota(jnp.int32, (_NH, _RD), 1) // _DH
    E = (he == de_i).astype(jnp.float32)
    ET = (lax.broadcasted_iota(jnp.int32, (_RD, _NH), 0) // _DH
          == lax.broadcasted_iota(jnp.int32, (_RD, _NH), 1)).astype(jnp.float32)
    s = _dot(p.reshape(_QT5 * _TOPK, _RD), ET).reshape(_QT5, _TOPK, _NH)
    s = s * (1.0 / math.sqrt(_DH))
    m = jnp.max(s, axis=1, keepdims=True)
    e = jnp.exp(s - m)
    a = e / jnp.sum(e, axis=1, keepdims=True)
    ab = _dot(a.reshape(_QT5 * _TOPK, _NH), E).reshape(_QT5, _TOPK, _RD)
    o = jnp.sum(ab * vg, axis=1)
    retr = _dot(o, mWo_ref[...]) + mbo_ref[...]
    comb = jnp.concatenate([de_ref[...], retr], axis=-1)
    z = jax.nn.relu(_ln(_dot(comb, W1_ref[...]) + b1_ref[...], g_ref[...], be_ref[...]))
    y_ref[...] = _dot(z, W2_ref[...]) + b2_ref[...]


def _att_call(de, qm, kvg, mWo, mbo, out_W1, out_b1, out_g, out_be, out_W2, out_b2):
    rows = de.shape[0]
    grid = (rows // _QT5,)

    def full(a):
        return pl.BlockSpec(a.shape, lambda i: (0,) * a.ndim)

    consts = [mWo, mbo, out_W1, out_b1, out_g, out_be, out_W2, out_b2]
    return pl.pallas_call(
        _att_body,
        grid=grid,
        in_specs=[pl.BlockSpec((_QT5, _RD), lambda i: (i, 0)),
                  pl.BlockSpec((_QT5, _RD), lambda i: (i, 0)),
                  pl.BlockSpec((_QT5, _TOPK, _RD), lambda i: (i, 0, 0))]
                 + [full(c) for c in consts],
        out_specs=pl.BlockSpec((_QT5, _HOR * _OD), lambda i: (i, 0)),
        out_shape=jax.ShapeDtypeStruct((rows, _HOR * _OD), jnp.float32),
    )(de, qm, kvg, *consts)


def kernel(history_data, W_temp, b_temp, spatial_emb, enc_W1, enc_b1, enc_g1,
           enc_be1, enc_W2, enc_b2, enc_g2, enc_be2, W_h2e, b_h2e, Wq, bq,
           Wk, bk, Wv, bv, mWq, mbq, mWk, mbk, mWv, mbv, mWo, mbo,
           out_W1, out_b1, out_g, out_be, out_W2, out_b2, store):
    x2 = history_data.transpose(0, 2, 1, 3).reshape(_R, _L * _C)
    spf = jnp.broadcast_to(spatial_emb[None], (_B, _N, _SD)).reshape(_R, _SD)

    r1 = lambda v: v.reshape(1, -1)
    r3 = lambda v: v.reshape(_EL, 1, -1)

    de, q, qm = _encoder_call(
        x2, spf, W_temp, r1(b_temp), enc_W1, r3(enc_b1), r3(enc_g1), r3(enc_be1),
        enc_W2, r3(enc_b2), r3(enc_g2), r3(enc_be2), W_h2e, r1(b_h2e),
        Wq, r1(bq), mWq, r1(mbq))

    kv = _kv_call(store, Wk, r1(bk), mWk, r1(mbk), Wv, r1(bv), mWv, r1(mbv))

    pad = jnp.zeros((_BTP - _BT,), jnp.int32)
    ys = []
    for s in range(_NS):
        sl = slice(s * _RS, (s + 1) * _RS)
        idx = _topk_call(q[sl], store)
        g = _gather_call(kv, jnp.concatenate([idx.reshape(_BT), pad]))[:_BT]
        ys.append(_att_call(de[sl], qm[sl], g.reshape(_RS, _TOPK, _RD),
                            mWo, r1(mbo), out_W1, r1(out_b1), r1(out_g),
                            r1(out_be), out_W2, r1(out_b2)))
    y = jnp.concatenate(ys, axis=0)

    return y.reshape(_B, _N, _HOR, _OD).transpose(0, 2, 1, 3)


# speedup vs baseline: 5.0412x; 1.0010x over previous
import functools
import math

import jax
import jax.numpy as jnp
from jax import lax
from jax.experimental import pallas as pl
from jax.experimental.pallas import tpu as pltpu
from jax.experimental.pallas import tpu_sc as plsc

_B, _L, _N, _C = 32, 12, 325, 3
_HOR, _OD = 12, 1
_TD, _SD = 64, 32
_F = _TD + _SD
_ED, _RD = 512, 512
_NH = 8
_DH = _RD // _NH
_KDOCS = 4096
_TOPK = 8
_EL = 3
_EPS = 1e-5
_R = _B * _N

_HI = lax.Precision.HIGHEST


def _ln(x, g, b):
    m = x.mean(-1, keepdims=True)
    v = ((x - m) ** 2).mean(-1, keepdims=True)
    return (x - m) * lax.rsqrt(v + _EPS) * g + b


def _dot(a, b):
    return jnp.dot(a, b, preferred_element_type=jnp.float32)


_QT1 = 400


def _enc_body(x_ref, sp_ref, Wt_ref, bt_ref, W1_ref, b1_ref, g1_ref, be1_ref,
              W2_ref, b2_ref, g2_ref, be2_ref, Wh_ref, bh_ref, Wq_ref, bq_ref,
              mWq_ref, mbq_ref, de_ref, q_ref, qm_ref):
    t = _dot(x_ref[...], Wt_ref[...]) + bt_ref[...]
    h = jnp.concatenate([t, sp_ref[...]], axis=-1)
    for i in range(_EL):
        h = jax.nn.relu(_ln(_dot(h, W1_ref[i]) + b1_ref[i], g1_ref[i], be1_ref[i]))
        h = jax.nn.relu(_ln(_dot(h, W2_ref[i]) + b2_ref[i], g2_ref[i], be2_ref[i]))
    de = _dot(h, Wh_ref[...]) + bh_ref[...]
    q = _dot(de, Wq_ref[...]) + bq_ref[...]
    de_ref[...] = de
    q_ref[...] = q
    qm_ref[...] = _dot(q, mWq_ref[...]) + mbq_ref[...]


def _encoder_call(x2, spf, W_temp, b_temp, enc_W1, enc_b1, enc_g1, enc_be1,
                  enc_W2, enc_b2, enc_g2, enc_be2, W_h2e, b_h2e, Wq, bq, mWq, mbq):
    grid = (_R // _QT1,)
    row = pl.BlockSpec((_QT1, None), lambda i: (i, 0))

    def full(a):
        return pl.BlockSpec(a.shape, lambda i: (0,) * a.ndim)

    row_specs = [pl.BlockSpec((_QT1, x2.shape[1]), lambda i: (i, 0)),
                 pl.BlockSpec((_QT1, spf.shape[1]), lambda i: (i, 0))]
    consts = [W_temp, b_temp, enc_W1, enc_b1, enc_g1, enc_be1,
              enc_W2, enc_b2, enc_g2, enc_be2, W_h2e, b_h2e, Wq, bq, mWq, mbq]
    out_spec = pl.BlockSpec((_QT1, _RD), lambda i: (i, 0))
    return pl.pallas_call(
        _enc_body,
        grid=grid,
        in_specs=row_specs + [full(c) for c in consts],
        out_specs=[out_spec, out_spec, out_spec],
        out_shape=[jax.ShapeDtypeStruct((_R, _RD), jnp.float32)] * 3,
    )(x2, spf, *consts)


def _kv_body(st_ref, Wk_ref, bk_ref, mWk_ref, mbk_ref,
             Wv_ref, bv_ref, mWv_ref, mbv_ref, kv_ref):
    s = st_ref[...]
    kk = _dot(_dot(s, Wk_ref[...]) + bk_ref[...], mWk_ref[...]) + mbk_ref[...]
    vv = _dot(_dot(s, Wv_ref[...]) + bv_ref[...], mWv_ref[...]) + mbv_ref[...]
    k16 = lax.bitcast_convert_type(kk.astype(jnp.bfloat16), jnp.uint16).astype(jnp.uint32)
    v16 = lax.bitcast_convert_type(vv.astype(jnp.bfloat16), jnp.uint16).astype(jnp.uint32)
    kv_ref[...] = lax.bitcast_convert_type(k16 | (v16 << 16), jnp.int32)


def _kv_call(store, Wk, bk, mWk, mbk, Wv, bv, mWv, mbv):
    T = 512
    grid = (_KDOCS // T,)

    def full(a):
        return pl.BlockSpec(a.shape, lambda i: (0,) * a.ndim)

    consts = [Wk, bk, mWk, mbk, Wv, bv, mWv, mbv]
    return pl.pallas_call(
        _kv_body,
        grid=grid,
        in_specs=[pl.BlockSpec((T, _RD), lambda i: (i, 0))] + [full(c) for c in consts],
        out_specs=pl.BlockSpec((T, _RD), lambda i: (i, 0)),
        out_shape=jax.ShapeDtypeStruct((_KDOCS, _RD), jnp.int32),
    )(store, *consts)


_NS = 2
_RS = _R // _NS
_QT3 = 200


def _topk_body(q_ref, st_ref, idx_ref):
    s = lax.dot_general(q_ref[...], st_ref[...], (((1,), (1,)), ((), ())),
                        preferred_element_type=jnp.float32)
    cols = lax.broadcasted_iota(jnp.int32, (_QT3, _KDOCS), 1)
    outs = []
    for _ in range(_TOPK):
        m = jnp.max(s, axis=1, keepdims=True)
        hit = s == m
        cj = jnp.min(jnp.where(hit, cols, _KDOCS), axis=1, keepdims=True)
        outs.append(cj)
        s = jnp.where(hit, -jnp.inf, s)
    idx_ref[...] = jnp.concatenate(outs, axis=1)


def _topk_call(q, store):
    rows = q.shape[0]
    grid = (rows // _QT3,)
    return pl.pallas_call(
        _topk_body,
        grid=grid,
        in_specs=[pl.BlockSpec((_QT3, _RD), lambda i: (i, 0)),
                  pl.BlockSpec((_KDOCS, _RD), lambda i: (0, 0))],
        out_specs=pl.BlockSpec((_QT3, _TOPK), lambda i: (i, 0)),
        out_shape=jax.ShapeDtypeStruct((rows, _TOPK), jnp.int32),
    )(q, store)


_NW = 32
_BT = _RS * _TOPK
_BTP = 43008
_BPW = _BTP // _NW
_CH = 112


_NCH = _BPW // _CH


@functools.cache
def _build_sc_gather():
    @functools.partial(
        pl.kernel,
        mesh=plsc.VectorSubcoreMesh(core_axis_name="c", subcore_axis_name="s"),
        out_type=jax.ShapeDtypeStruct((_BTP, _RD), jnp.int32),
        scratch_types=[
            pltpu.VMEM((_BPW,), jnp.int32),
            pltpu.VMEM((2, _CH, _RD), jnp.int32),
        ] + [pltpu.SemaphoreType.DMA] * 4,
    )
    def _sc_gather(table_hbm, idx_hbm, out_hbm, idx_v, rows_v, *sems):
        gsem = sems[:2]
        wsem = sems[2:]
        wid = lax.axis_index("s") * 2 + lax.axis_index("c")
        base = wid * _BPW
        pltpu.sync_copy(idx_hbm.at[pl.ds(base, _BPW)], idx_v)

        def g_start(i, b):
            for sub in range(_CH // 16):
                idx16 = idx_v[pl.ds(i * _CH + sub * 16, 16)]
                pltpu.make_async_copy(
                    table_hbm.at[idx16],
                    rows_v.at[b, pl.ds(sub * 16, 16)], gsem[b]).start()

        def g_wait(i, b):
            for sub in range(_CH // 16):
                pltpu.make_async_copy(
                    table_hbm.at[idx_v[pl.ds(sub * 16, 16)]],
                    rows_v.at[b, pl.ds(sub * 16, 16)], gsem[b]).wait()

        def w_desc(i, b):
            return pltpu.make_async_copy(
                rows_v.at[b], out_hbm.at[pl.ds(base + i * _CH, _CH)], wsem[b])

        g_start(0, 0)
        g_start(1, 1)

        def body(j, carry):
            i0 = 2 * j
            i1 = 2 * j + 1
            g_wait(i0, 0)
            w_desc(i0, 0).start()
            w_desc(i0, 0).wait()
            g_start(i0 + 2, 0)
            g_wait(i1, 1)
            w_desc(i1, 1).start()
            w_desc(i1, 1).wait()
            g_start(i1 + 2, 1)
            return carry

        lax.fori_loop(0, _NCH // 2 - 1, body, 0)
        g_wait(_NCH - 2, 0)
        w_desc(_NCH - 2, 0).start()
        g_wait(_NCH - 1, 1)
        w_desc(_NCH - 1, 1).start()
        w_desc(_NCH - 2, 0).wait()
        w_desc(_NCH - 1, 1).wait()

    return _sc_gather


def _gather_call(kv, idx_flat):
    return _build_sc_gather()(kv, idx_flat)


_QT5 = 200


def _att_body(de_ref, qm_ref, kv_ref, mWo_ref, mbo_ref, W1_ref, b1_ref,
              g_ref, be_ref, W2_ref, b2_ref, y_ref):
    qm = qm_ref[...]
    u = lax.bitcast_convert_type(kv_ref[...], jnp.uint32)
    kg = lax.bitcast_convert_type(
        (u & 0xffff).astype(jnp.uint16), jnp.bfloat16).astype(jnp.float32)
    vg = lax.bitcast_convert_type(
        (u >> 16).astype(jnp.uint16), jnp.bfloat16).astype(jnp.float32)
    p = kg * qm[:, None, :]
    he = lax.broadcasted_iota(jnp.int32, (_NH, _RD), 0)
    de_i = lax.broadcasted_iota(jnp.int32, (_NH, _RD), 1) // _DH
    E = (he == de_i).astype(jnp.float32)
    ET = (lax.broadcasted_iota(jnp.int32, (_RD, _NH), 0) // _DH
          == lax.broadcasted_iota(jnp.int32, (_RD, _NH), 1)).astype(jnp.float32)
    s = _dot(p.reshape(_QT5 * _TOPK, _RD), ET).reshape(_QT5, _TOPK, _NH)
    s = s * (1.0 / math.sqrt(_DH))
    m = jnp.max(s, axis=1, keepdims=True)
    e = jnp.exp(s - m)
    a = e / jnp.sum(e, axis=1, keepdims=True)
    ab = _dot(a.reshape(_QT5 * _TOPK, _NH), E).reshape(_QT5, _TOPK, _RD)
    o = jnp.sum(ab * vg, axis=1)
    retr = _dot(o, mWo_ref[...]) + mbo_ref[...]
    comb = jnp.concatenate([de_ref[...], retr], axis=-1)
    z = jax.nn.relu(_ln(_dot(comb, W1_ref[...]) + b1_ref[...], g_ref[...], be_ref[...]))
    y_ref[...] = _dot(z, W2_ref[...]) + b2_ref[...]


def _att_call(de, qm, kvg, mWo, mbo, out_W1, out_b1, out_g, out_be, out_W2, out_b2):
    rows = de.shape[0]
    grid = (rows // _QT5,)

    def full(a):
        return pl.BlockSpec(a.shape, lambda i: (0,) * a.ndim)

    consts = [mWo, mbo, out_W1, out_b1, out_g, out_be, out_W2, out_b2]
    return pl.pallas_call(
        _att_body,
        grid=grid,
        in_specs=[pl.BlockSpec((_QT5, _RD), lambda i: (i, 0)),
                  pl.BlockSpec((_QT5, _RD), lambda i: (i, 0)),
                  pl.BlockSpec((_QT5, _TOPK, _RD), lambda i: (i, 0, 0))]
                 + [full(c) for c in consts],
        out_specs=pl.BlockSpec((_QT5, _HOR * _OD), lambda i: (i, 0)),
        out_shape=jax.ShapeDtypeStruct((rows, _HOR * _OD), jnp.float32),
    )(de, qm, kvg, *consts)


def kernel(history_data, W_temp, b_temp, spatial_emb, enc_W1, enc_b1, enc_g1,
           enc_be1, enc_W2, enc_b2, enc_g2, enc_be2, W_h2e, b_h2e, Wq, bq,
           Wk, bk, Wv, bv, mWq, mbq, mWk, mbk, mWv, mbv, mWo, mbo,
           out_W1, out_b1, out_g, out_be, out_W2, out_b2, store):
    x2 = history_data.transpose(0, 2, 1, 3).reshape(_R, _L * _C)
    spf = jnp.broadcast_to(spatial_emb[None], (_B, _N, _SD)).reshape(_R, _SD)

    r1 = lambda v: v.reshape(1, -1)
    r3 = lambda v: v.reshape(_EL, 1, -1)

    de, q, qm = _encoder_call(
        x2, spf, W_temp, r1(b_temp), enc_W1, r3(enc_b1), r3(enc_g1), r3(enc_be1),
        enc_W2, r3(enc_b2), r3(enc_g2), r3(enc_be2), W_h2e, r1(b_h2e),
        Wq, r1(bq), mWq, r1(mbq))

    kv = _kv_call(store, Wk, r1(bk), mWk, r1(mbk), Wv, r1(bv), mWv, r1(mbv))

    pad = jnp.zeros((_BTP - _BT,), jnp.int32)
    ys = []
    for s in range(_NS):
        sl = slice(s * _RS, (s + 1) * _RS)
        idx = _topk_call(q[sl], store)
        g = _gather_call(kv, jnp.concatenate([idx.reshape(_BT), pad]))[:_BT]
        ys.append(_att_call(de[sl], qm[sl], g.reshape(_RS, _TOPK, _RD),
                            mWo, r1(mbo), out_W1, r1(out_b1), r1(out_g),
                            r1(out_be), out_W2, r1(out_b2)))
    y = jnp.concatenate(ys, axis=0)

    return y.reshape(_B, _N, _HOR, _OD).transpose(0, 2, 1, 3)
